# Initial kernel scaffold; baseline (speedup 1.0000x reference)
import jax
import jax.numpy as jnp
from jax.experimental import pallas as pl


def kernel(x, edge_index, dual_x, dual_edge_index, epoch, W1, att_src1, att_dst1, b1, W2, att_src2, att_dst2, b2, Wg1, bg1, Wg2, bg2):
    raise NotImplementedError("write your pallas kernel here")


# speedup vs baseline: 2.1624x; 2.1624x over previous
import dataclasses
import functools

import jax
import jax.numpy as jnp
from jax import lax
from jax.experimental import pallas as pl
from jax.experimental.pallas import tpu as pltpu
from jax.experimental.pallas import tpu_sc as plsc

N = 10000
E = 160000
F_IN = 128
H = 8
HID = 64
CLS = 16
ND = 160000
ED = 1000000

NDP = 161792
NDH = NDP // 2
NDQ = NDP // 4
NDQS = NDQ // 16
NP = 10112
NPH = NP // 2
NPHS = NPH // 16
EDP = 1003520
EP = 163840

_mesh = plsc.VectorSubcoreMesh(core_axis_name="c", subcore_axis_name="s",
                               num_cores=2, num_subcores=16)

_cp = pltpu.CompilerParams()
_flds = pltpu.CompilerParams.__dataclass_fields__
if "needs_layout_passes" in _flds:
    _cp = dataclasses.replace(_cp, needs_layout_passes=False)
if "use_tc_tiling_on_sc" in _flds:
    _cp = dataclasses.replace(_cp, use_tc_tiling_on_sc=False)

_NEG1 = -1


def _leaky(x):
    return jnp.where(x >= 0, x, 0.2 * x)


def _fill_stripe(buf, shared, row0, nrows, zrows):
    done = 0
    while done < nrows:
        n = min(zrows, nrows - done)
        pltpu.sync_copy(buf.at[pl.ds(0, n)],
                        shared.at[pl.ds(row0 + done, n)])
        done += n


def _dump_stripe(shared, out_hbm, buf, srow0, drow0, nrows, zrows):
    done = 0
    while done < nrows:
        n = min(zrows, nrows - done)
        pltpu.sync_copy(shared.at[pl.ds(srow0 + done, n)],
                        buf.at[pl.ds(0, n)])
        pltpu.sync_copy(buf.at[pl.ds(0, n)],
                        out_hbm.at[pl.ds(drow0 + done, n)])
        done += n


def _mask_pair(sidx, didx, msrc, mdst, lo16, nh16, neg16):
    for o in range(8):
        ob = o * 16
        si = sidx[pl.ds(ob, 16)]
        di = didx[pl.ds(ob, 16)]
        local = di - lo16
        keep = jnp.logical_and(local >= 0, local < nh16)
        msrc[pl.ds(ob, 16)] = jnp.where(keep, si, neg16)
        mdst[pl.ds(ob, 16)] = jnp.where(keep, local, neg16)


def _sc_hist(dstp, ones128, zeros1d):
    @functools.partial(
        pl.kernel,
        out_type=jax.ShapeDtypeStruct((NDP,), jnp.float32),
        mesh=_mesh,
        compiler_params=_cp,
        scratch_types=[
            pltpu.VMEM((128,), jnp.int32),
            pltpu.VMEM((128,), jnp.int32),
            pltpu.VMEM((128,), jnp.float32),
            pltpu.VMEM((2048,), jnp.float32),
            pltpu.VMEM_SHARED((NDH,), jnp.float32),
            pltpu.SemaphoreType.DMA,
        ],
    )
    def k(dst_hbm, ones_hbm, z_hbm, out_hbm, didx, mdst, ones_v, zb, acc,
          sem):
        c = lax.axis_index("c")
        s = lax.axis_index("s")
        lo16 = lax.broadcast(c * NDH, (16,))
        nh16 = jnp.full((16,), NDH, jnp.int32)
        neg16 = jnp.full((16,), _NEG1, jnp.int32)
        pltpu.sync_copy(ones_hbm, ones_v)
        pltpu.sync_copy(z_hbm, zb)
        _fill_stripe(zb, acc, s * (NDH // 16), NDH // 16, 2048)
        plsc.subcore_barrier()

        @pl.loop(0, EDP // 16 // 128)
        def _(g):
            base = s * (EDP // 16) + g * 128
            pltpu.sync_copy(dst_hbm.at[pl.ds(base, 128)], didx)
            for o in range(8):
                ob = o * 16
                di = didx[pl.ds(ob, 16)]
                local = di - lo16
                keep = jnp.logical_and(local >= 0, local < nh16)
                mdst[pl.ds(ob, 16)] = jnp.where(keep, local, neg16)
            pltpu.sync_copy(
                ones_v, acc.at[plsc.Indices(mdst, ignored_value=_NEG1)],
                add=True)

        plsc.subcore_barrier()
        _dump_stripe(acc, out_hbm, zb, s * (NDH // 16),
                     c * NDH + s * (NDH // 16), NDH // 16, 2048)

    return k(dstp, ones128, zeros1d)


def _sc_segsum(srcp, dstp, tables, zeros2d):
    nchunks = len(tables)

    @functools.partial(
        pl.kernel,
        out_type=[jax.ShapeDtypeStruct((NDP, 8), jnp.float32)
                  for _ in range(nchunks)],
        mesh=_mesh,
        compiler_params=_cp,
        scratch_types=[
            pltpu.VMEM((128,), jnp.int32),
            pltpu.VMEM((128,), jnp.int32),
            pltpu.VMEM((128,), jnp.int32),
            pltpu.VMEM((128,), jnp.int32),
            pltpu.VMEM((128, 8), jnp.float32),
            pltpu.VMEM((128, 8), jnp.float32),
            pltpu.VMEM((128, 8), jnp.float32),
            pltpu.VMEM_SHARED((NDQ, 8), jnp.float32),
            pltpu.SemaphoreType.DMA,
        ],
    )
    def k(src_hbm, dst_hbm, *rest):
        tabs = rest[:nchunks]
        z_hbm = rest[nchunks]
        outs = rest[nchunks + 1:2 * nchunks + 1]
        sidx, didx, msrc, mdst, rows, zb, tb, acc, sem = \
            rest[2 * nchunks + 1:]
        c = lax.axis_index("c")
        s = lax.axis_index("s")
        nh16 = jnp.full((16,), NDQ, jnp.int32)
        neg16 = jnp.full((16,), _NEG1, jnp.int32)
        pltpu.sync_copy(z_hbm.at[pl.ds(0, 128)], zb)
        for p in range(nchunks):
            for q in range(2):
                lo = c * NDH + q * NDQ
                lo16 = lax.broadcast(lo, (16,))
                _fill_stripe(zb, acc, s * NDQS, NDQS, 128)
                plsc.subcore_barrier()

                @pl.loop(0, EDP // 16 // 128)
                def _(g, p=p):
                    base = s * (EDP // 16) + g * 128
                    pltpu.sync_copy(src_hbm.at[pl.ds(base, 128)], sidx)
                    pltpu.sync_copy(dst_hbm.at[pl.ds(base, 128)], didx)
                    _mask_pair(sidx, didx, msrc, mdst, lo16, nh16, neg16)
                    pltpu.async_copy(
                        tabs[p].at[plsc.Indices(msrc, ignored_value=_NEG1)],
                        rows, sem).wait()
                    pltpu.sync_copy(
                        rows,
                        acc.at[plsc.Indices(mdst, ignored_value=_NEG1)],
                        add=True)

                plsc.subcore_barrier()
                _dump_stripe(acc, outs[p], tb, s * NDQS, lo + s * NDQS,
                             NDQS, 128)
                plsc.subcore_barrier()

    return k(srcp, dstp, *tables, zeros2d)


def _sc_gat1_attn(srcp, dstp, asrcT, adstT, bb, zeros1d):
    @functools.partial(
        pl.kernel,
        out_type=[jax.ShapeDtypeStruct((H * EP,), jnp.float32),
                  jax.ShapeDtypeStruct((2 * NP * 4,), jnp.float32)],
        mesh=_mesh,
        compiler_params=_cp,
        scratch_types=[
            pltpu.VMEM((128,), jnp.int32),
            pltpu.VMEM((128,), jnp.int32),
            pltpu.VMEM((128,), jnp.int32),
            pltpu.VMEM((128,), jnp.float32),
            pltpu.VMEM((512,), jnp.float32),
            pltpu.VMEM((NP,), jnp.float32),
            pltpu.VMEM((NP,), jnp.float32),
            pltpu.VMEM((H, 16), jnp.float32),
            pltpu.VMEM_SHARED((NP * 4,), jnp.float32),
            pltpu.SemaphoreType.DMA,
        ],
    )
    def k(src_hbm, dst_hbm, asrc_hbm, adst_hbm, bb_hbm, z_hbm,
          ex_hbm, s1_hbm, sidx, didx, midx, exbuf, zb, sa, da, bbuf, sacc,
          sem):
        c = lax.axis_index("c")
        s = lax.axis_index("s")
        pltpu.sync_copy(bb_hbm, bbuf)
        pltpu.sync_copy(z_hbm.at[pl.ds(0, 512)], zb)
        _fill_stripe(zb, sacc, s * (NP * 4 // 16), NP * 4 // 16, 512)
        plsc.subcore_barrier()
        for hd in range(H):
            @pl.when(c == hd % 2)
            def _(hd=hd):
                pltpu.sync_copy(asrc_hbm.at[pl.ds(hd * NP, NP)], sa)
                pltpu.sync_copy(adst_hbm.at[pl.ds(hd * NP, NP)], da)
                vb = bbuf[hd, :]

                @pl.loop(0, 80)
                def _(g):
                    base = s * (EP // 16) + g * 128
                    pltpu.sync_copy(src_hbm.at[pl.ds(base, 128)], sidx)
                    pltpu.sync_copy(dst_hbm.at[pl.ds(base, 128)], didx)
                    for o in range(8):
                        ob = o * 16
                        si = sidx[pl.ds(ob, 16)]
                        di = didx[pl.ds(ob, 16)]
                        va = plsc.load_gather(sa, [si])
                        vd = plsc.load_gather(da, [di])
                        ex = jnp.exp(_leaky(va + vd) - vb)
                        exbuf[pl.ds(ob, 16)] = ex
                        midx[pl.ds(ob, 16)] = di * 4 + (hd // 2)
                    pltpu.sync_copy(exbuf, sacc.at[midx], add=True)
                    pltpu.sync_copy(
                        exbuf, ex_hbm.at[pl.ds(hd * EP + base, 128)])

        plsc.subcore_barrier()
        _dump_stripe(sacc, s1_hbm, zb, s * (NP * 4 // 16),
                     c * NP * 4 + s * (NP * 4 // 16), NP * 4 // 16, 512)

    return k(srcp, dstp, asrcT, adstT, bb, zeros1d)


def _sc_gat1_num(srcp, dstp, ex1, hchunks, zeros2d):
    @functools.partial(
        pl.kernel,
        out_type=[jax.ShapeDtypeStruct((NP, 64), jnp.float32)
                  for _ in range(8)],
        mesh=_mesh,
        compiler_params=_cp,
        scratch_types=[
            pltpu.VMEM((128,), jnp.int32),
            pltpu.VMEM((128,), jnp.int32),
            pltpu.VMEM((128,), jnp.int32),
            pltpu.VMEM((128,), jnp.int32),
            pltpu.VMEM((128,), jnp.float32),
            pltpu.VMEM((128, 64), jnp.float32),
            pltpu.VMEM((79, 64), jnp.float32),
            pltpu.VMEM_SHARED((NPH, 64), jnp.float32),
            pltpu.SemaphoreType.DMA,
        ],
    )
    def k(src_hbm, dst_hbm, ex_hbm, h0, h1, h2, h3, h4, h5, h6, h7, z_hbm,
          o0, o1, o2, o3, o4, o5, o6, o7,
          sidx, didx, msrc, mdst, exb, rows, tb, acc, sem):
        hb = [h0, h1, h2, h3, h4, h5, h6, h7]
        outs = [o0, o1, o2, o3, o4, o5, o6, o7]
        c = lax.axis_index("c")
        s = lax.axis_index("s")
        lo16 = lax.broadcast(c * NPH, (16,))
        nh16 = jnp.full((16,), NPH, jnp.int32)
        neg16 = jnp.full((16,), _NEG1, jnp.int32)
        iotas = [lax.iota(jnp.int32, 16) + o * 16 for o in range(8)]
        for hd in range(8):
            pltpu.sync_copy(z_hbm, tb)
            _fill_stripe(tb, acc, s * NPHS, NPHS, 79)
            plsc.subcore_barrier()

            @pl.loop(0, 80)
            def _(g, hd=hd):
                base = s * (EP // 16) + g * 128
                pltpu.sync_copy(src_hbm.at[pl.ds(base, 128)], sidx)
                pltpu.sync_copy(dst_hbm.at[pl.ds(base, 128)], didx)
                pltpu.sync_copy(
                    ex_hbm.at[pl.ds(hd * EP + base, 128)], exb)
                _mask_pair(sidx, didx, msrc, mdst, lo16, nh16, neg16)
                pltpu.async_copy(
                    hb[hd].at[plsc.Indices(msrc, ignored_value=_NEG1)],
                    rows, sem).wait()

                @pl.loop(0, 64)
                def _(f):
                    fs = lax.broadcast(f, (16,))
                    for o in range(8):
                        v = plsc.load_gather(rows, [iotas[o], fs])
                        plsc.store_scatter(rows, [iotas[o], fs],
                                           v * exb[pl.ds(o * 16, 16)])

                pltpu.sync_copy(
                    rows, acc.at[plsc.Indices(mdst, ignored_value=_NEG1)],
                    add=True)

            plsc.subcore_barrier()
            _dump_stripe(acc, outs[hd], tb, s * NPHS, c * NPH + s * NPHS,
                         NPHS, 79)
            plsc.subcore_barrier()

    return k(srcp, dstp, ex1, *hchunks, zeros2d)


def _sc_gat2(srcp, dstp, asrc2, adst2, b2b, h2g, zeros2d, zeros632):
    @functools.partial(
        pl.kernel,
        out_type=[jax.ShapeDtypeStruct((NP, CLS), jnp.float32),
                  jax.ShapeDtypeStruct((NP,), jnp.float32)],
        mesh=_mesh,
        compiler_params=_cp,
        scratch_types=[
            pltpu.VMEM((128,), jnp.int32),
            pltpu.VMEM((128,), jnp.int32),
            pltpu.VMEM((128,), jnp.int32),
            pltpu.VMEM((128,), jnp.int32),
            pltpu.VMEM((128,), jnp.float32),
            pltpu.VMEM((632,), jnp.float32),
            pltpu.VMEM((NP,), jnp.float32),
            pltpu.VMEM((NP,), jnp.float32),
            pltpu.VMEM((1, 16), jnp.float32),
            pltpu.VMEM((128, CLS), jnp.float32),
            pltpu.VMEM((158, CLS), jnp.float32),
            pltpu.VMEM_SHARED((NPH, CLS), jnp.float32),
            pltpu.VMEM_SHARED((NPH,), jnp.float32),
            pltpu.SemaphoreType.DMA,
        ],
    )
    def k(src_hbm, dst_hbm, sa_hbm, da_hbm, bb_hbm, tab_hbm,
          z_hbm, z632_hbm, num_hbm, s2_hbm,
          sidx, didx, msrc, mdst, exbuf, zb1, sa, da, bbuf, rows, tb, acc,
          s2sh, sem):
        c = lax.axis_index("c")
        s = lax.axis_index("s")
        lo16 = lax.broadcast(c * NPH, (16,))
        nh16 = jnp.full((16,), NPH, jnp.int32)
        neg16 = jnp.full((16,), _NEG1, jnp.int32)
        pltpu.sync_copy(sa_hbm, sa)
        pltpu.sync_copy(da_hbm, da)
        pltpu.sync_copy(bb_hbm, bbuf)
        pltpu.sync_copy(z632_hbm, zb1)
        vb = bbuf[0, :]
        iotas = [lax.iota(jnp.int32, 16) + o * 16 for o in range(8)]

        pltpu.sync_copy(z_hbm, tb)
        _fill_stripe(tb, acc, s * NPHS, NPHS, 158)

        @pl.when(s < 8)
        def _():
            pltpu.sync_copy(zb1, s2sh.at[pl.ds(s * 632, 632)])

        plsc.subcore_barrier()

        @pl.loop(0, 80)
        def _(g):
            base = s * (EP // 16) + g * 128
            pltpu.sync_copy(src_hbm.at[pl.ds(base, 128)], sidx)
            pltpu.sync_copy(dst_hbm.at[pl.ds(base, 128)], didx)
            _mask_pair(sidx, didx, msrc, mdst, lo16, nh16, neg16)
            pltpu.async_copy(
                tab_hbm.at[plsc.Indices(msrc, ignored_value=_NEG1)],
                rows, sem).wait()
            for o in range(8):
                ob = o * 16
                si = sidx[pl.ds(ob, 16)]
                di = didx[pl.ds(ob, 16)]
                va = plsc.load_gather(sa, [si])
                vd = plsc.load_gather(da, [di])
                ex = jnp.exp(_leaky(va + vd) - vb)
                exbuf[pl.ds(ob, 16)] = ex
            pltpu.sync_copy(
                exbuf, s2sh.at[plsc.Indices(mdst, ignored_value=_NEG1)],
                add=True)

            @pl.loop(0, CLS)
            def _(f):
                fs = lax.broadcast(f, (16,))
                for o in range(8):
                    v = plsc.load_gather(rows, [iotas[o], fs])
                    plsc.store_scatter(rows, [iotas[o], fs],
                                       v * exbuf[pl.ds(o * 16, 16)])

            pltpu.sync_copy(
                rows, acc.at[plsc.Indices(mdst, ignored_value=_NEG1)],
                add=True)

        plsc.subcore_barrier()
        _dump_stripe(acc, num_hbm, tb, s * NPHS, c * NPH + s * NPHS,
                     NPHS, 158)

        @pl.when(s < 8)
        def _():
            pltpu.sync_copy(s2sh.at[pl.ds(s * 632, 632)], zb1)
            pltpu.sync_copy(zb1,
                            s2_hbm.at[pl.ds(c * NPH + s * 632, 632)])

    return k(srcp, dstp, asrc2, adst2, b2b, h2g, zeros2d, zeros632)


def _t1(dual_x, Wg1, dinv2d):
    BLK = 3200

    def body(x_ref, w_ref, dinv_ref, *g_refs):
        g = dinv_ref[...] * jnp.dot(x_ref[...], w_ref[...],
                                    preferred_element_type=jnp.float32)
        for p in range(8):
            g_refs[p][...] = g[:, 8 * p:8 * p + 8]

    return pl.pallas_call(
        body,
        grid=(ND // BLK,),
        in_specs=[
            pl.BlockSpec((BLK, 4), lambda i: (i, 0)),
            pl.BlockSpec((4, 64), lambda i: (0, 0)),
            pl.BlockSpec((BLK, 1), lambda i: (i, 0)),
        ],
        out_specs=[pl.BlockSpec((BLK, 8), lambda i: (i, 0))
                   for _ in range(8)],
        out_shape=[jax.ShapeDtypeStruct((ND, 8), jnp.float32)
                   for _ in range(8)],
    )(dual_x, Wg1, dinv2d)


def _t2(dual_x, Wg1, Wg2, bg1, dinv2d, s1):
    BLK = 3200

    def body(x_ref, w1_ref, w2_ref, b1_ref, dinv_ref, *rest):
        s_refs = rest[:8]
        g20_ref, g21_ref, g2f_ref = rest[8:]
        dinv = dinv_ref[...]
        g1 = dinv * jnp.dot(x_ref[...], w1_ref[...],
                            preferred_element_type=jnp.float32)
        S = jnp.concatenate([s_refs[p][...] for p in range(8)], axis=1)
        q1 = dinv * (S + g1) + b1_ref[0, :]
        h2 = jnp.dot(jnp.maximum(q1, 0.0), w2_ref[...],
                     preferred_element_type=jnp.float32)
        g2 = dinv * h2
        g20_ref[...] = g2[:, :8]
        g21_ref[...] = g2[:, 8:]
        g2f_ref[...] = g2

    return pl.pallas_call(
        body,
        grid=(ND // BLK,),
        in_specs=[
            pl.BlockSpec((BLK, 4), lambda i: (i, 0)),
            pl.BlockSpec((4, 64), lambda i: (0, 0)),
            pl.BlockSpec((64, 16), lambda i: (0, 0)),
            pl.BlockSpec((1, 64), lambda i: (0, 0)),
            pl.BlockSpec((BLK, 1), lambda i: (i, 0)),
        ] + [pl.BlockSpec((BLK, 8), lambda i: (i, 0)) for _ in range(8)],
        out_specs=[pl.BlockSpec((BLK, 8), lambda i: (i, 0)),
                   pl.BlockSpec((BLK, 8), lambda i: (i, 0)),
                   pl.BlockSpec((BLK, 16), lambda i: (i, 0))],
        out_shape=[jax.ShapeDtypeStruct((ND, 8), jnp.float32),
                   jax.ShapeDtypeStruct((ND, 8), jnp.float32),
                   jax.ShapeDtypeStruct((ND, 16), jnp.float32)],
    )(dual_x, Wg1, Wg2, bg1.reshape(1, 64), dinv2d, *s1)


def _t3(dinv2d, s2, g2f, bg2):
    BLK = 3200

    def body(dinv_ref, s0_ref, s1_ref, g_ref, b_ref, out_ref):
        S = jnp.concatenate([s0_ref[...], s1_ref[...]], axis=1)
        out_ref[...] = dinv_ref[...] * (S + g_ref[...]) + b_ref[0, :]

    return pl.pallas_call(
        body,
        grid=(ND // BLK,),
        in_specs=[
            pl.BlockSpec((BLK, 1), lambda i: (i, 0)),
            pl.BlockSpec((BLK, 8), lambda i: (i, 0)),
            pl.BlockSpec((BLK, 8), lambda i: (i, 0)),
            pl.BlockSpec((BLK, 16), lambda i: (i, 0)),
            pl.BlockSpec((1, 16), lambda i: (0, 0)),
        ],
        out_specs=pl.BlockSpec((BLK, 16), lambda i: (i, 0)),
        out_shape=jax.ShapeDtypeStruct((ND, 16), jnp.float32),
    )(dinv2d, s2[0], s2[1], g2f, bg2.reshape(1, 16))


def _t4(x, W1, att_src1, att_dst1):
    BLK = 1000

    def body(*refs):
        x_ref, w_ref, as_ref, ad_ref = refs[:4]
        hrefs = refs[4:12]
        at_ref, dt_ref, ms_ref, md_ref = refs[12:]
        i = pl.program_id(0)
        h = jnp.dot(x_ref[...], w_ref[...],
                    preferred_element_type=jnp.float32)

        @pl.when(i == 0)
        def _():
            ms_ref[...] = jnp.full((H, 128), -jnp.inf, jnp.float32)
            md_ref[...] = jnp.full((H, 128), -jnp.inf, jnp.float32)

        for hd in range(H):
            hcol = h[:, 64 * hd:64 * hd + 64]
            hrefs[hd][...] = hcol
            va = jnp.dot(hcol, as_ref[hd, :],
                         preferred_element_type=jnp.float32)
            vd = jnp.dot(hcol, ad_ref[hd, :],
                         preferred_element_type=jnp.float32)
            at_ref[:, hd:hd + 1] = va[:, None]
            dt_ref[:, hd:hd + 1] = vd[:, None]
            ms_ref[hd, :] = jnp.maximum(ms_ref[hd, :], jnp.max(va))
            md_ref[hd, :] = jnp.maximum(md_ref[hd, :], jnp.max(vd))

    return pl.pallas_call(
        body,
        grid=(N // BLK,),
        in_specs=[
            pl.BlockSpec((BLK, F_IN), lambda i: (i, 0)),
            pl.BlockSpec((F_IN, H * HID), lambda i: (0, 0)),
            pl.BlockSpec((H, HID), lambda i: (0, 0)),
            pl.BlockSpec((H, HID), lambda i: (0, 0)),
        ],
        out_specs=[pl.BlockSpec((BLK, 64), lambda i: (i, 0))
                   for _ in range(8)] +
                  [pl.BlockSpec((BLK, H), lambda i: (i, 0)),
                   pl.BlockSpec((BLK, H), lambda i: (i, 0)),
                   pl.BlockSpec((H, 128), lambda i: (0, 0)),
                   pl.BlockSpec((H, 128), lambda i: (0, 0))],
        out_shape=[jax.ShapeDtypeStruct((N, 64), jnp.float32)
                   for _ in range(8)] +
                  [jax.ShapeDtypeStruct((N, H), jnp.float32),
                   jax.ShapeDtypeStruct((N, H), jnp.float32),
                   jax.ShapeDtypeStruct((H, 128), jnp.float32),
                   jax.ShapeDtypeStruct((H, 128), jnp.float32)],
    )(x, W1, att_src1, att_dst1)


def _t5(nums, hs, s1parts, asrc, adst, bb, b1, W2, att_src2, att_dst2):
    BLK = 1000

    def body(*refs):
        nrefs = refs[:8]
        hrefs = refs[8:16]
        (s1_ref, at_ref, dt_ref, bb_ref, b1_ref, w2_ref, as2_ref, ad2_ref,
         hg_ref, a2_ref, d2_ref, m2s_ref, m2d_ref) = refs[16:]
        i = pl.program_id(0)
        cols = []
        for hd in range(H):
            exs = jnp.exp(_leaky(at_ref[:, hd:hd + 1] + dt_ref[:, hd:hd + 1])
                          - bb_ref[hd, 0])
            den = (s1_ref[hd % 2, :, hd // 2:hd // 2 + 1] + exs + 1e-16)
            cols.append((nrefs[hd][...] + exs * hrefs[hd][...]) / den)
        out1 = jnp.concatenate(cols, axis=1) + b1_ref[0, :]
        hh = jnp.where(out1 > 0, out1, jnp.exp(jnp.minimum(out1, 0.0)) - 1.0)
        hg = jnp.dot(hh, w2_ref[...], preferred_element_type=jnp.float32)
        a2 = jnp.dot(hg, as2_ref[0, :], preferred_element_type=jnp.float32)
        d2 = jnp.dot(hg, ad2_ref[0, :], preferred_element_type=jnp.float32)
        hg_ref[...] = hg
        a2_ref[...] = a2[:, None]
        d2_ref[...] = d2[:, None]

        @pl.when(i == 0)
        def _():
            m2s_ref[...] = jnp.full((8, 128), -jnp.inf, jnp.float32)
            md = jnp.full((8, 128), -jnp.inf, jnp.float32)
            m2d_ref[...] = md

        m2s_ref[...] = jnp.maximum(m2s_ref[...], jnp.max(a2))
        m2d_ref[...] = jnp.maximum(m2d_ref[...], jnp.max(d2))

    return pl.pallas_call(
        body,
        grid=(N // BLK,),
        in_specs=[pl.BlockSpec((BLK, 64), lambda i: (i, 0))
                  for _ in range(8)] +
                 [pl.BlockSpec((BLK, 64), lambda i: (i, 0))
                  for _ in range(8)] +
                 [pl.BlockSpec((2, BLK, 4), lambda i: (0, i, 0)),
                  pl.BlockSpec((BLK, H), lambda i: (i, 0)),
                  pl.BlockSpec((BLK, H), lambda i: (i, 0)),
                  pl.BlockSpec((H, 16), lambda i: (0, 0)),
                  pl.BlockSpec((1, 512), lambda i: (0, 0)),
                  pl.BlockSpec((512, CLS), lambda i: (0, 0)),
                  pl.BlockSpec((1, CLS), lambda i: (0, 0)),
                  pl.BlockSpec((1, CLS), lambda i: (0, 0))],
        out_specs=[pl.BlockSpec((BLK, CLS), lambda i: (i, 0)),
                   pl.BlockSpec((BLK, 1), lambda i: (i, 0)),
                   pl.BlockSpec((BLK, 1), lambda i: (i, 0)),
                   pl.BlockSpec((8, 128), lambda i: (0, 0)),
                   pl.BlockSpec((8, 128), lambda i: (0, 0))],
        out_shape=[jax.ShapeDtypeStruct((N, CLS), jnp.float32),
                   jax.ShapeDtypeStruct((N, 1), jnp.float32),
                   jax.ShapeDtypeStruct((N, 1), jnp.float32),
                   jax.ShapeDtypeStruct((8, 128), jnp.float32),
                   jax.ShapeDtypeStruct((8, 128), jnp.float32)],
    )(*nums, *hs, s1parts, asrc, adst, bb, b1.reshape(1, 512), W2,
      att_src2, att_dst2)


def _t6(num2, s2, h2g, asrc2, adst2, b2b, b2):
    BLK = 2000

    def body(n_ref, s_ref, hg_ref, a2_ref, d2_ref, bb_ref, b2_ref, out_ref):
        exs = jnp.exp(_leaky(a2_ref[...] + d2_ref[...]) - bb_ref[0, 0])
        den = s_ref[...] + exs + 1e-16
        out_ref[...] = (n_ref[...] + exs * hg_ref[...]) / den + b2_ref[0, :]

    return pl.pallas_call(
        body,
        grid=(N // BLK,),
        in_specs=[
            pl.BlockSpec((BLK, CLS), lambda i: (i, 0)),
            pl.BlockSpec((BLK, 1), lambda i: (i, 0)),
            pl.BlockSpec((BLK, CLS), lambda i: (i, 0)),
            pl.BlockSpec((BLK, 1), lambda i: (i, 0)),
            pl.BlockSpec((BLK, 1), lambda i: (i, 0)),
            pl.BlockSpec((1, 16), lambda i: (0, 0)),
            pl.BlockSpec((1, CLS), lambda i: (0, 0)),
        ],
        out_specs=pl.BlockSpec((BLK, CLS), lambda i: (i, 0)),
        out_shape=jax.ShapeDtypeStruct((N, CLS), jnp.float32),
    )(num2, s2, h2g, asrc2, adst2, b2b, b2.reshape(1, CLS))


def kernel(x, edge_index, dual_x, dual_edge_index, epoch,
           W1, att_src1, att_dst1, b1, W2, att_src2, att_dst2, b2,
           Wg1, bg1, Wg2, bg2):
    f32 = jnp.float32
    i32 = jnp.int32

    dsrc = jnp.concatenate(
        [dual_edge_index[0], jnp.zeros((EDP - ED,), i32)])
    ddst = jnp.concatenate(
        [dual_edge_index[1], jnp.full((EDP - ED,), ND, i32)])
    psrc = jnp.concatenate([edge_index[0], jnp.zeros((EP - E,), i32)])
    pdst = jnp.concatenate([edge_index[1], jnp.full((EP - E,), N, i32)])

    ones128 = jnp.ones((128,), f32)
    z1d = jnp.zeros((2048,), f32)
    z2d8 = jnp.zeros((128, 8), f32)
    z2d64 = jnp.zeros((79, 64), f32)
    z2dc = jnp.zeros((158, CLS), f32)
    z632 = jnp.zeros((632,), f32)

    histp = _sc_hist(ddst, ones128, z1d)
    deg = histp[:ND] + 1.0
    dinv2d = lax.rsqrt(deg)[:, None]
    g1 = _t1(dual_x, Wg1, dinv2d)
    s1 = []
    for j in range(4):
        s1 += _sc_segsum(dsrc, ddst, [g1[2 * j], g1[2 * j + 1]], z2d8)
    s1 = [a[:ND] for a in s1]
    g20, g21, g2f = _t2(dual_x, Wg1, Wg2, bg1, dinv2d, s1)
    s2 = _sc_segsum(dsrc, ddst, [g20, g21], z2d8)
    s2 = [a[:ND] for a in s2]
    Q = _t3(dinv2d, s2, g2f, bg2)

    t4 = _t4(x, W1, att_src1, att_dst1)
    hs = list(t4[0:8])
    asrc, adst, msrc, mdst = t4[8], t4[9], t4[10], t4[11]
    B = _leaky(jnp.max(msrc, axis=1) + jnp.max(mdst, axis=1))
    bb = jnp.broadcast_to(B[:, None], (H, 16)).astype(f32)
    asrcTp = jnp.pad(asrc.T, ((0, 0), (0, NP - N))).reshape(H * NP)
    adstTp = jnp.pad(adst.T, ((0, 0), (0, NP - N))).reshape(H * NP)
    ex1, s1flat = _sc_gat1_attn(psrc, pdst, asrcTp, adstTp, bb, z1d)
    s1parts = s1flat.reshape(2, NP, 4)[:, :N, :]
    nums = _sc_gat1_num(psrc, pdst, ex1, hs, z2d64)
    nums = [a[:N] for a in nums]
    h2g, asrc2, adst2, m2s, m2d = _t5(
        nums, hs, s1parts, asrc, adst, bb, b1, W2, att_src2, att_dst2)
    B2 = _leaky(jnp.max(m2s) + jnp.max(m2d))
    b2b = jnp.broadcast_to(B2, (1, 16)).astype(f32)
    a2p = jnp.pad(asrc2.reshape(N), (0, NP - N))
    d2p = jnp.pad(adst2.reshape(N), (0, NP - N))
    num2, s2g = _sc_gat2(psrc, pdst, a2p, d2p, b2b, h2g, z2dc, z632)
    out = _t6(num2[:N], s2g[:N].reshape(N, 1), h2g, asrc2, adst2, b2b, b2)
    return (out, Q)


# speedup vs baseline: 2.4726x; 1.1434x over previous
import dataclasses
import functools

import jax
import jax.numpy as jnp
from jax import lax
from jax.experimental import pallas as pl
from jax.experimental.pallas import tpu as pltpu
from jax.experimental.pallas import tpu_sc as plsc

N = 10000
E = 160000
F_IN = 128
H = 8
HID = 64
CLS = 16
ND = 160000
ED = 1000000

NDP = 161792
NDH = NDP // 2
NDQ = NDP // 4
NDQS = NDQ // 16
NP = 10112
NPH = NP // 2
NPHS = NPH // 16
NP4 = NP // 4
NP4S = NP4 // 16
EDP = 1003520
EP = 163840

_mesh = plsc.VectorSubcoreMesh(core_axis_name="c", subcore_axis_name="s",
                               num_cores=2, num_subcores=16)

_cp = pltpu.CompilerParams()
_flds = pltpu.CompilerParams.__dataclass_fields__
if "needs_layout_passes" in _flds:
    _cp = dataclasses.replace(_cp, needs_layout_passes=False)
if "use_tc_tiling_on_sc" in _flds:
    _cp = dataclasses.replace(_cp, use_tc_tiling_on_sc=False)

_NEG1 = -1


def _leaky(x):
    return jnp.where(x >= 0, x, 0.2 * x)


def _fill_stripe(buf, shared, row0, nrows, zrows):
    done = 0
    while done < nrows:
        n = min(zrows, nrows - done)
        pltpu.sync_copy(buf.at[pl.ds(0, n)],
                        shared.at[pl.ds(row0 + done, n)])
        done += n


def _dump_stripe(shared, out_hbm, buf, srow0, drow0, nrows, zrows):
    done = 0
    while done < nrows:
        n = min(zrows, nrows - done)
        pltpu.sync_copy(shared.at[pl.ds(srow0 + done, n)],
                        buf.at[pl.ds(0, n)])
        pltpu.sync_copy(buf.at[pl.ds(0, n)],
                        out_hbm.at[pl.ds(drow0 + done, n)])
        done += n


def _mask_pair(sidx, didx, msrc, mdst, lo16, nh16, neg16):
    for o in range(8):
        ob = o * 16
        si = sidx[pl.ds(ob, 16)]
        di = didx[pl.ds(ob, 16)]
        local = di - lo16
        keep = jnp.logical_and(local >= 0, local < nh16)
        msrc[pl.ds(ob, 16)] = jnp.where(keep, si, neg16)
        mdst[pl.ds(ob, 16)] = jnp.where(keep, local, neg16)


def _sc_hist(dstp, ones128, zeros1d):
    @functools.partial(
        pl.kernel,
        out_type=jax.ShapeDtypeStruct((NDP,), jnp.float32),
        mesh=_mesh,
        compiler_params=_cp,
        scratch_types=[
            pltpu.VMEM((128,), jnp.int32),
            pltpu.VMEM((128,), jnp.int32),
            pltpu.VMEM((128,), jnp.float32),
            pltpu.VMEM((512,), jnp.float32),
            pltpu.VMEM_SHARED((NDQ,), jnp.float32),
            pltpu.SemaphoreType.DMA,
        ],
    )
    def k(dst_hbm, ones_hbm, z_hbm, out_hbm, didx, mdst, ones_v, zb, acc,
          sem):
        c = lax.axis_index("c")
        s = lax.axis_index("s")
        nh16 = jnp.full((16,), NDQ, jnp.int32)
        neg16 = jnp.full((16,), _NEG1, jnp.int32)
        pltpu.sync_copy(ones_hbm, ones_v)
        for q in range(2):
            lo = c * NDH + q * NDQ
            lo16 = lax.broadcast(lo, (16,))
            pltpu.sync_copy(z_hbm.at[pl.ds(0, 512)], zb)
            _fill_stripe(zb, acc, s * NDQS, NDQS, 512)
            plsc.subcore_barrier()

            @pl.loop(0, EDP // 16 // 128)
            def _(g):
                base = s * (EDP // 16) + g * 128
                pltpu.sync_copy(dst_hbm.at[pl.ds(base, 128)], didx)
                for o in range(8):
                    ob = o * 16
                    di = didx[pl.ds(ob, 16)]
                    local = di - lo16
                    keep = jnp.logical_and(local >= 0, local < nh16)
                    mdst[pl.ds(ob, 16)] = jnp.where(keep, local, neg16)
                pltpu.sync_copy(
                    ones_v,
                    acc.at[plsc.Indices(mdst, ignored_value=_NEG1)],
                    add=True)

            plsc.subcore_barrier()
            _dump_stripe(acc, out_hbm, zb, s * NDQS, lo + s * NDQS,
                         NDQS, 512)
            plsc.subcore_barrier()

    return k(dstp, ones128, zeros1d)


def _sc_segsum(srcp, dstp, table, zeros2d):
    NSTR = 5
    B = NSTR * 128

    @functools.partial(
        pl.kernel,
        out_type=jax.ShapeDtypeStruct((NDP, 16), jnp.float32),
        mesh=_mesh,
        compiler_params=_cp,
        scratch_types=[
            pltpu.VMEM((B,), jnp.int32),
            pltpu.VMEM((B,), jnp.int32),
            pltpu.VMEM((NSTR, 128), jnp.int32),
            pltpu.VMEM((NSTR, 128), jnp.int32),
            pltpu.VMEM((B, 16), jnp.float32),
            pltpu.VMEM_SHARED((NDQ, 16), jnp.float32),
            pltpu.SemaphoreType.DMA,
            pltpu.SemaphoreType.DMA,
        ],
    )
    def k(src_hbm, dst_hbm, tab, z_hbm, out, sidx, didx, msrc, mdst, rows,
          acc, sem, sem2):
        c = lax.axis_index("c")
        s = lax.axis_index("s")
        nh16 = jnp.full((16,), NDQ, jnp.int32)
        neg16 = jnp.full((16,), _NEG1, jnp.int32)
        for q in range(2):
            lo = c * NDH + q * NDQ
            lo16 = lax.broadcast(lo, (16,))
            pltpu.sync_copy(z_hbm, rows.at[pl.ds(0, 128)])
            _fill_stripe(rows, acc, s * NDQS, NDQS, 128)
            plsc.subcore_barrier()

            @pl.loop(0, EDP // 16 // B)
            def _(g):
                base = s * (EDP // 16) + g * B
                pltpu.sync_copy(src_hbm.at[pl.ds(base, B)], sidx)
                pltpu.sync_copy(dst_hbm.at[pl.ds(base, B)], didx)
                for j in range(NSTR):
                    for o in range(8):
                        ob = j * 128 + o * 16
                        si = sidx[pl.ds(ob, 16)]
                        di = didx[pl.ds(ob, 16)]
                        local = di - lo16
                        keep = jnp.logical_and(local >= 0, local < nh16)
                        msrc[j, pl.ds(o * 16, 16)] = \
                            jnp.where(keep, si, neg16)
                        mdst[j, pl.ds(o * 16, 16)] = \
                            jnp.where(keep, local, neg16)
                for j in range(NSTR):
                    pltpu.async_copy(
                        tab.at[plsc.Indices(msrc.at[j],
                                            ignored_value=_NEG1)],
                        rows.at[pl.ds(j * 128, 128)], sem).wait()
                    pltpu.sync_copy(
                        rows.at[pl.ds(j * 128, 128)],
                        acc.at[plsc.Indices(mdst.at[j],
                                            ignored_value=_NEG1)],
                        add=True)

            plsc.subcore_barrier()
            _dump_stripe(acc, out, rows, s * NDQS, lo + s * NDQS,
                         NDQS, 128)
            plsc.subcore_barrier()

    return k(srcp, dstp, table, zeros2d)


def _sc_gat1_attn(srcp, dstp, asrcT, adstT, bb, zeros1d):
    @functools.partial(
        pl.kernel,
        out_type=[jax.ShapeDtypeStruct((H * EP,), jnp.float32),
                  jax.ShapeDtypeStruct((2 * NP * 4,), jnp.float32)],
        mesh=_mesh,
        compiler_params=_cp,
        scratch_types=[
            pltpu.VMEM((128,), jnp.int32),
            pltpu.VMEM((128,), jnp.int32),
            pltpu.VMEM((128,), jnp.int32),
            pltpu.VMEM((128,), jnp.float32),
            pltpu.VMEM((512,), jnp.float32),
            pltpu.VMEM((NP,), jnp.float32),
            pltpu.VMEM((NP,), jnp.float32),
            pltpu.VMEM((H, 16), jnp.float32),
            pltpu.VMEM_SHARED((NP * 4,), jnp.float32),
            pltpu.SemaphoreType.DMA,
        ],
    )
    def k(src_hbm, dst_hbm, asrc_hbm, adst_hbm, bb_hbm, z_hbm,
          ex_hbm, s1_hbm, sidx, didx, midx, exbuf, zb, sa, da, bbuf, sacc,
          sem):
        c = lax.axis_index("c")
        s = lax.axis_index("s")
        pltpu.sync_copy(bb_hbm, bbuf)
        pltpu.sync_copy(z_hbm.at[pl.ds(0, 512)], zb)
        _fill_stripe(zb, sacc, s * (NP * 4 // 16), NP * 4 // 16, 512)
        plsc.subcore_barrier()
        for hd in range(H):
            @pl.when(c == hd % 2)
            def _(hd=hd):
                pltpu.sync_copy(asrc_hbm.at[pl.ds(hd * NP, NP)], sa)
                pltpu.sync_copy(adst_hbm.at[pl.ds(hd * NP, NP)], da)
                vb = bbuf[hd, :]

                @pl.loop(0, 80)
                def _(g):
                    base = s * (EP // 16) + g * 128
                    pltpu.sync_copy(src_hbm.at[pl.ds(base, 128)], sidx)
                    pltpu.sync_copy(dst_hbm.at[pl.ds(base, 128)], didx)
                    for o in range(8):
                        ob = o * 16
                        si = sidx[pl.ds(ob, 16)]
                        di = didx[pl.ds(ob, 16)]
                        va = plsc.load_gather(sa, [si])
                        vd = plsc.load_gather(da, [di])
                        ex = jnp.exp(_leaky(va + vd) - vb)
                        exbuf[pl.ds(ob, 16)] = ex
                        midx[pl.ds(ob, 16)] = di * 4 + (hd // 2)
                    pltpu.sync_copy(exbuf, sacc.at[midx], add=True)
                    pltpu.sync_copy(
                        exbuf, ex_hbm.at[pl.ds(hd * EP + base, 128)])

        plsc.subcore_barrier()
        _dump_stripe(sacc, s1_hbm, zb, s * (NP * 4 // 16),
                     c * NP * 4 + s * (NP * 4 // 16), NP * 4 // 16, 512)

    return k(srcp, dstp, asrcT, adstT, bb, zeros1d)


def _sc_gat1_num(srcp, dstp, ex1, hchunks, zeros2d):
    @functools.partial(
        pl.kernel,
        out_type=[jax.ShapeDtypeStruct((NP, 64), jnp.float32)
                  for _ in range(8)],
        mesh=_mesh,
        compiler_params=_cp,
        scratch_types=[
            pltpu.VMEM((128,), jnp.int32),
            pltpu.VMEM((128,), jnp.int32),
            pltpu.VMEM((128,), jnp.int32),
            pltpu.VMEM((128,), jnp.int32),
            pltpu.VMEM((128,), jnp.float32),
            pltpu.VMEM((128, 64), jnp.float32),
            pltpu.VMEM_SHARED((NP4, 64), jnp.float32),
            pltpu.SemaphoreType.DMA,
        ],
    )
    def k(src_hbm, dst_hbm, ex_hbm, h0, h1, h2, h3, h4, h5, h6, h7, z_hbm,
          o0, o1, o2, o3, o4, o5, o6, o7,
          sidx, didx, msrc, mdst, exb, rows, acc, sem):
        hb = [h0, h1, h2, h3, h4, h5, h6, h7]
        outs = [o0, o1, o2, o3, o4, o5, o6, o7]
        c = lax.axis_index("c")
        s = lax.axis_index("s")
        nh16 = jnp.full((16,), NP4, jnp.int32)
        neg16 = jnp.full((16,), _NEG1, jnp.int32)
        iotas = [lax.iota(jnp.int32, 16) + o * 16 for o in range(8)]
        for hd in range(8):
            for q in range(2):
                lo = c * NPH + q * NP4
                lo16 = lax.broadcast(lo, (16,))
                pltpu.sync_copy(z_hbm, rows)
                _fill_stripe(rows, acc, s * NP4S, NP4S, 128)
                plsc.subcore_barrier()

                @pl.loop(0, 80)
                def _(g, hd=hd):
                    base = s * (EP // 16) + g * 128
                    pltpu.sync_copy(src_hbm.at[pl.ds(base, 128)], sidx)
                    pltpu.sync_copy(dst_hbm.at[pl.ds(base, 128)], didx)
                    pltpu.sync_copy(
                        ex_hbm.at[pl.ds(hd * EP + base, 128)], exb)
                    _mask_pair(sidx, didx, msrc, mdst, lo16, nh16, neg16)
                    pltpu.async_copy(
                        hb[hd].at[plsc.Indices(msrc, ignored_value=_NEG1)],
                        rows, sem).wait()

                    @pl.loop(0, 64)
                    def _(f):
                        fs = lax.broadcast(f, (16,))
                        for o in range(8):
                            v = plsc.load_gather(rows, [iotas[o], fs])
                            plsc.store_scatter(rows, [iotas[o], fs],
                                               v * exb[pl.ds(o * 16, 16)])

                    pltpu.sync_copy(
                        rows,
                        acc.at[plsc.Indices(mdst, ignored_value=_NEG1)],
                        add=True)

                plsc.subcore_barrier()
                _dump_stripe(acc, outs[hd], rows, s * NP4S,
                             lo + s * NP4S, NP4S, 128)
                plsc.subcore_barrier()

    return k(srcp, dstp, ex1, *hchunks, zeros2d)


def _sc_gat2(srcp, dstp, asrc2, adst2, b2b, h2g, zeros2d, zeros632):
    @functools.partial(
        pl.kernel,
        out_type=[jax.ShapeDtypeStruct((NP, CLS), jnp.float32),
                  jax.ShapeDtypeStruct((NP,), jnp.float32)],
        mesh=_mesh,
        compiler_params=_cp,
        scratch_types=[
            pltpu.VMEM((128,), jnp.int32),
            pltpu.VMEM((128,), jnp.int32),
            pltpu.VMEM((128,), jnp.int32),
            pltpu.VMEM((128,), jnp.int32),
            pltpu.VMEM((128,), jnp.int32),
            pltpu.VMEM((128,), jnp.float32),
            pltpu.VMEM((632,), jnp.float32),
            pltpu.VMEM((NP,), jnp.float32),
            pltpu.VMEM((NP,), jnp.float32),
            pltpu.VMEM((1, 16), jnp.float32),
            pltpu.VMEM((128, CLS), jnp.float32),
            pltpu.VMEM_SHARED((NP4, CLS), jnp.float32),
            pltpu.VMEM_SHARED((NPH,), jnp.float32),
            pltpu.SemaphoreType.DMA,
        ],
    )
    def k(src_hbm, dst_hbm, sa_hbm, da_hbm, bb_hbm, tab_hbm,
          z_hbm, z632_hbm, num_hbm, s2_hbm,
          sidx, didx, msrc, mdst, ms2, exbuf, zb1, sa, da, bbuf, rows, acc,
          s2sh, sem):
        c = lax.axis_index("c")
        s = lax.axis_index("s")
        hlo16 = lax.broadcast(c * NPH, (16,))
        hnh16 = jnp.full((16,), NPH, jnp.int32)
        nh16 = jnp.full((16,), NP4, jnp.int32)
        neg16 = jnp.full((16,), _NEG1, jnp.int32)
        pltpu.sync_copy(sa_hbm, sa)
        pltpu.sync_copy(da_hbm, da)
        pltpu.sync_copy(bb_hbm, bbuf)
        pltpu.sync_copy(z632_hbm, zb1)
        vb = bbuf[0, :]
        iotas = [lax.iota(jnp.int32, 16) + o * 16 for o in range(8)]

        @pl.when(s < 8)
        def _():
            pltpu.sync_copy(zb1, s2sh.at[pl.ds(s * 632, 632)])

        for q in range(2):
            lo = c * NPH + q * NP4
            lo16 = lax.broadcast(lo, (16,))
            pltpu.sync_copy(z_hbm, rows)
            _fill_stripe(rows, acc, s * NP4S, NP4S, 128)
            plsc.subcore_barrier()

            @pl.loop(0, 80)
            def _(g, q=q):
                base = s * (EP // 16) + g * 128
                pltpu.sync_copy(src_hbm.at[pl.ds(base, 128)], sidx)
                pltpu.sync_copy(dst_hbm.at[pl.ds(base, 128)], didx)
                _mask_pair(sidx, didx, msrc, mdst, lo16, nh16, neg16)
                pltpu.async_copy(
                    tab_hbm.at[plsc.Indices(msrc, ignored_value=_NEG1)],
                    rows, sem).wait()
                for o in range(8):
                    ob = o * 16
                    si = sidx[pl.ds(ob, 16)]
                    di = didx[pl.ds(ob, 16)]
                    va = plsc.load_gather(sa, [si])
                    vd = plsc.load_gather(da, [di])
                    ex = jnp.exp(_leaky(va + vd) - vb)
                    exbuf[pl.ds(ob, 16)] = ex
                    if q == 0:
                        local = di - hlo16
                        keep = jnp.logical_and(local >= 0, local < hnh16)
                        ms2[pl.ds(ob, 16)] = jnp.where(keep, local, neg16)
                if q == 0:
                    pltpu.sync_copy(
                        exbuf,
                        s2sh.at[plsc.Indices(ms2, ignored_value=_NEG1)],
                        add=True)

                @pl.loop(0, CLS)
                def _(f):
                    fs = lax.broadcast(f, (16,))
                    for o in range(8):
                        v = plsc.load_gather(rows, [iotas[o], fs])
                        plsc.store_scatter(rows, [iotas[o], fs],
                                           v * exbuf[pl.ds(o * 16, 16)])

                pltpu.sync_copy(
                    rows, acc.at[plsc.Indices(mdst, ignored_value=_NEG1)],
                    add=True)

            plsc.subcore_barrier()
            _dump_stripe(acc, num_hbm, rows, s * NP4S, lo + s * NP4S,
                         NP4S, 128)
            plsc.subcore_barrier()

        @pl.when(s < 8)
        def _():
            pltpu.sync_copy(s2sh.at[pl.ds(s * 632, 632)], zb1)
            pltpu.sync_copy(zb1,
                            s2_hbm.at[pl.ds(c * NPH + s * 632, 632)])

    return k(srcp, dstp, asrc2, adst2, b2b, h2g, zeros2d, zeros632)


def _t1(dual_x, Wg1, dinv2d):
    BLK = 3200

    def body(x_ref, w_ref, dinv_ref, *g_refs):
        g = dinv_ref[...] * jnp.dot(x_ref[...], w_ref[...],
                                    preferred_element_type=jnp.float32)
        for p in range(4):
            g_refs[p][...] = g[:, 16 * p:16 * p + 16]

    return pl.pallas_call(
        body,
        grid=(ND // BLK,),
        in_specs=[
            pl.BlockSpec((BLK, 4), lambda i: (i, 0)),
            pl.BlockSpec((4, 64), lambda i: (0, 0)),
            pl.BlockSpec((BLK, 1), lambda i: (i, 0)),
        ],
        out_specs=[pl.BlockSpec((BLK, 16), lambda i: (i, 0))
                   for _ in range(4)],
        out_shape=[jax.ShapeDtypeStruct((ND, 16), jnp.float32)
                   for _ in range(4)],
    )(dual_x, Wg1, dinv2d)


def _t2(dual_x, Wg1, Wg2, bg1, dinv2d, s1):
    BLK = 3200

    def body(x_ref, w1_ref, w2_ref, b1_ref, dinv_ref, *rest):
        s_refs = rest[:4]
        g2f_ref = rest[4]
        dinv = dinv_ref[...]
        g1 = dinv * jnp.dot(x_ref[...], w1_ref[...],
                            preferred_element_type=jnp.float32)
        S = jnp.concatenate([s_refs[p][...] for p in range(4)], axis=1)
        q1 = dinv * (S + g1) + b1_ref[0, :]
        h2 = jnp.dot(jnp.maximum(q1, 0.0), w2_ref[...],
                     preferred_element_type=jnp.float32)
        g2f_ref[...] = dinv * h2

    return pl.pallas_call(
        body,
        grid=(ND // BLK,),
        in_specs=[
            pl.BlockSpec((BLK, 4), lambda i: (i, 0)),
            pl.BlockSpec((4, 64), lambda i: (0, 0)),
            pl.BlockSpec((64, 16), lambda i: (0, 0)),
            pl.BlockSpec((1, 64), lambda i: (0, 0)),
            pl.BlockSpec((BLK, 1), lambda i: (i, 0)),
        ] + [pl.BlockSpec((BLK, 16), lambda i: (i, 0)) for _ in range(4)],
        out_specs=pl.BlockSpec((BLK, 16), lambda i: (i, 0)),
        out_shape=jax.ShapeDtypeStruct((ND, 16), jnp.float32),
    )(dual_x, Wg1, Wg2, bg1.reshape(1, 64), dinv2d, *s1)


def _t3(dinv2d, s2, g2f, bg2):
    BLK = 3200

    def body(dinv_ref, s_ref, g_ref, b_ref, out_ref):
        out_ref[...] = (dinv_ref[...] * (s_ref[...] + g_ref[...])
                        + b_ref[0, :])

    return pl.pallas_call(
        body,
        grid=(ND // BLK,),
        in_specs=[
            pl.BlockSpec((BLK, 1), lambda i: (i, 0)),
            pl.BlockSpec((BLK, 16), lambda i: (i, 0)),
            pl.BlockSpec((BLK, 16), lambda i: (i, 0)),
            pl.BlockSpec((1, 16), lambda i: (0, 0)),
        ],
        out_specs=pl.BlockSpec((BLK, 16), lambda i: (i, 0)),
        out_shape=jax.ShapeDtypeStruct((ND, 16), jnp.float32),
    )(dinv2d, s2, g2f, bg2.reshape(1, 16))


def _t4(x, W1, att_src1, att_dst1):
    BLK = 1000

    def body(*refs):
        x_ref, w_ref, as_ref, ad_ref = refs[:4]
        hrefs = refs[4:12]
        at_ref, dt_ref, ms_ref, md_ref = refs[12:]
        i = pl.program_id(0)
        h = jnp.dot(x_ref[...], w_ref[...],
                    preferred_element_type=jnp.float32)

        @pl.when(i == 0)
        def _():
            ms_ref[...] = jnp.full((H, 128), -jnp.inf, jnp.float32)
            md_ref[...] = jnp.full((H, 128), -jnp.inf, jnp.float32)

        for hd in range(H):
            hcol = h[:, 64 * hd:64 * hd + 64]
            hrefs[hd][...] = hcol
            va = jnp.dot(hcol, as_ref[hd, :],
                         preferred_element_type=jnp.float32)
            vd = jnp.dot(hcol, ad_ref[hd, :],
                         preferred_element_type=jnp.float32)
            at_ref[:, hd:hd + 1] = va[:, None]
            dt_ref[:, hd:hd + 1] = vd[:, None]
            ms_ref[hd, :] = jnp.maximum(ms_ref[hd, :], jnp.max(va))
            md_ref[hd, :] = jnp.maximum(md_ref[hd, :], jnp.max(vd))

    return pl.pallas_call(
        body,
        grid=(N // BLK,),
        in_specs=[
            pl.BlockSpec((BLK, F_IN), lambda i: (i, 0)),
            pl.BlockSpec((F_IN, H * HID), lambda i: (0, 0)),
            pl.BlockSpec((H, HID), lambda i: (0, 0)),
            pl.BlockSpec((H, HID), lambda i: (0, 0)),
        ],
        out_specs=[pl.BlockSpec((BLK, 64), lambda i: (i, 0))
                   for _ in range(8)] +
                  [pl.BlockSpec((BLK, H), lambda i: (i, 0)),
                   pl.BlockSpec((BLK, H), lambda i: (i, 0)),
                   pl.BlockSpec((H, 128), lambda i: (0, 0)),
                   pl.BlockSpec((H, 128), lambda i: (0, 0))],
        out_shape=[jax.ShapeDtypeStruct((N, 64), jnp.float32)
                   for _ in range(8)] +
                  [jax.ShapeDtypeStruct((N, H), jnp.float32),
                   jax.ShapeDtypeStruct((N, H), jnp.float32),
                   jax.ShapeDtypeStruct((H, 128), jnp.float32),
                   jax.ShapeDtypeStruct((H, 128), jnp.float32)],
    )(x, W1, att_src1, att_dst1)


def _t5(nums, hs, s1parts, asrc, adst, bb, b1, W2, att_src2, att_dst2):
    BLK = 1000

    def body(*refs):
        nrefs = refs[:8]
        hrefs = refs[8:16]
        (s1_ref, at_ref, dt_ref, bb_ref, b1_ref, w2_ref, as2_ref, ad2_ref,
         hg_ref, a2_ref, d2_ref, m2s_ref, m2d_ref) = refs[16:]
        i = pl.program_id(0)
        cols = []
        for hd in range(H):
            exs = jnp.exp(_leaky(at_ref[:, hd:hd + 1] + dt_ref[:, hd:hd + 1])
                          - bb_ref[hd, 0])
            den = (s1_ref[hd % 2, :, hd // 2:hd // 2 + 1] + exs + 1e-16)
            cols.append((nrefs[hd][...] + exs * hrefs[hd][...]) / den)
        out1 = jnp.concatenate(cols, axis=1) + b1_ref[0, :]
        hh = jnp.where(out1 > 0, out1, jnp.exp(jnp.minimum(out1, 0.0)) - 1.0)
        hg = jnp.dot(hh, w2_ref[...], preferred_element_type=jnp.float32)
        a2 = jnp.dot(hg, as2_ref[0, :], preferred_element_type=jnp.float32)
        d2 = jnp.dot(hg, ad2_ref[0, :], preferred_element_type=jnp.float32)
        hg_ref[...] = hg
        a2_ref[...] = a2[:, None]
        d2_ref[...] = d2[:, None]

        @pl.when(i == 0)
        def _():
            m2s_ref[...] = jnp.full((8, 128), -jnp.inf, jnp.float32)
            md = jnp.full((8, 128), -jnp.inf, jnp.float32)
            m2d_ref[...] = md

        m2s_ref[...] = jnp.maximum(m2s_ref[...], jnp.max(a2))
        m2d_ref[...] = jnp.maximum(m2d_ref[...], jnp.max(d2))

    return pl.pallas_call(
        body,
        grid=(N // BLK,),
        in_specs=[pl.BlockSpec((BLK, 64), lambda i: (i, 0))
                  for _ in range(8)] +
                 [pl.BlockSpec((BLK, 64), lambda i: (i, 0))
                  for _ in range(8)] +
                 [pl.BlockSpec((2, BLK, 4), lambda i: (0, i, 0)),
                  pl.BlockSpec((BLK, H), lambda i: (i, 0)),
                  pl.BlockSpec((BLK, H), lambda i: (i, 0)),
                  pl.BlockSpec((H, 16), lambda i: (0, 0)),
                  pl.BlockSpec((1, 512), lambda i: (0, 0)),
                  pl.BlockSpec((512, CLS), lambda i: (0, 0)),
                  pl.BlockSpec((1, CLS), lambda i: (0, 0)),
                  pl.BlockSpec((1, CLS), lambda i: (0, 0))],
        out_specs=[pl.BlockSpec((BLK, CLS), lambda i: (i, 0)),
                   pl.BlockSpec((BLK, 1), lambda i: (i, 0)),
                   pl.BlockSpec((BLK, 1), lambda i: (i, 0)),
                   pl.BlockSpec((8, 128), lambda i: (0, 0)),
                   pl.BlockSpec((8, 128), lambda i: (0, 0))],
        out_shape=[jax.ShapeDtypeStruct((N, CLS), jnp.float32),
                   jax.ShapeDtypeStruct((N, 1), jnp.float32),
                   jax.ShapeDtypeStruct((N, 1), jnp.float32),
                   jax.ShapeDtypeStruct((8, 128), jnp.float32),
                   jax.ShapeDtypeStruct((8, 128), jnp.float32)],
    )(*nums, *hs, s1parts, asrc, adst, bb, b1.reshape(1, 512), W2,
      att_src2, att_dst2)


def _t6(num2, s2, h2g, asrc2, adst2, b2b, b2):
    BLK = 2000

    def body(n_ref, s_ref, hg_ref, a2_ref, d2_ref, bb_ref, b2_ref, out_ref):
        exs = jnp.exp(_leaky(a2_ref[...] + d2_ref[...]) - bb_ref[0, 0])
        den = s_ref[...] + exs + 1e-16
        out_ref[...] = (n_ref[...] + exs * hg_ref[...]) / den + b2_ref[0, :]

    return pl.pallas_call(
        body,
        grid=(N // BLK,),
        in_specs=[
            pl.BlockSpec((BLK, CLS), lambda i: (i, 0)),
            pl.BlockSpec((BLK, 1), lambda i: (i, 0)),
            pl.BlockSpec((BLK, CLS), lambda i: (i, 0)),
            pl.BlockSpec((BLK, 1), lambda i: (i, 0)),
            pl.BlockSpec((BLK, 1), lambda i: (i, 0)),
            pl.BlockSpec((1, 16), lambda i: (0, 0)),
            pl.BlockSpec((1, CLS), lambda i: (0, 0)),
        ],
        out_specs=pl.BlockSpec((BLK, CLS), lambda i: (i, 0)),
        out_shape=jax.ShapeDtypeStruct((N, CLS), jnp.float32),
    )(num2, s2, h2g, asrc2, adst2, b2b, b2.reshape(1, CLS))


def kernel(x, edge_index, dual_x, dual_edge_index, epoch,
           W1, att_src1, att_dst1, b1, W2, att_src2, att_dst2, b2,
           Wg1, bg1, Wg2, bg2):
    f32 = jnp.float32
    i32 = jnp.int32

    dsrc = jnp.concatenate(
        [dual_edge_index[0], jnp.zeros((EDP - ED,), i32)])
    ddst = jnp.concatenate(
        [dual_edge_index[1], jnp.full((EDP - ED,), ND, i32)])
    psrc = jnp.concatenate([edge_index[0], jnp.zeros((EP - E,), i32)])
    pdst = jnp.concatenate([edge_index[1], jnp.full((EP - E,), N, i32)])

    ones128 = jnp.ones((128,), f32)
    z1d = jnp.zeros((2048,), f32)
    z2d16 = jnp.zeros((128, 16), f32)
    z2d64 = jnp.zeros((128, 64), f32)
    z632 = jnp.zeros((632,), f32)

    histp = _sc_hist(ddst, ones128, z1d)
    deg = histp[:ND] + 1.0
    dinv2d = lax.rsqrt(deg)[:, None]
    g1 = _t1(dual_x, Wg1, dinv2d)
    s1 = [_sc_segsum(dsrc, ddst, g1[j], z2d16)[:ND] for j in range(4)]
    g2f = _t2(dual_x, Wg1, Wg2, bg1, dinv2d, s1)
    s2 = _sc_segsum(dsrc, ddst, g2f, z2d16)[:ND]
    Q = _t3(dinv2d, s2, g2f, bg2)

    t4 = _t4(x, W1, att_src1, att_dst1)
    hs = list(t4[0:8])
    asrc, adst, msrc, mdst = t4[8], t4[9], t4[10], t4[11]
    B = _leaky(jnp.max(msrc, axis=1) + jnp.max(mdst, axis=1))
    bb = jnp.broadcast_to(B[:, None], (H, 16)).astype(f32)
    asrcTp = jnp.pad(asrc.T, ((0, 0), (0, NP - N))).reshape(H * NP)
    adstTp = jnp.pad(adst.T, ((0, 0), (0, NP - N))).reshape(H * NP)
    ex1, s1flat = _sc_gat1_attn(psrc, pdst, asrcTp, adstTp, bb, z1d)
    s1parts = s1flat.reshape(2, NP, 4)[:, :N, :]
    nums = _sc_gat1_num(psrc, pdst, ex1, hs, z2d64)
    nums = [a[:N] for a in nums]
    h2g, asrc2, adst2, m2s, m2d = _t5(
        nums, hs, s1parts, asrc, adst, bb, b1, W2, att_src2, att_dst2)
    B2 = _leaky(jnp.max(m2s) + jnp.max(m2d))
    b2b = jnp.broadcast_to(B2, (1, 16)).astype(f32)
    a2p = jnp.pad(asrc2.reshape(N), (0, NP - N))
    d2p = jnp.pad(adst2.reshape(N), (0, NP - N))
    num2, s2g = _sc_gat2(psrc, pdst, a2p, d2p, b2b, h2g, z2d16, z632)
    out = _t6(num2[:N], s2g[:N].reshape(N, 1), h2g, asrc2, adst2, b2b, b2)
    return (out, Q)


# speedup vs baseline: 2.7310x; 1.1045x over previous
import dataclasses
import functools

import jax
import jax.numpy as jnp
from jax import lax
from jax.experimental import pallas as pl
from jax.experimental.pallas import tpu as pltpu
from jax.experimental.pallas import tpu_sc as plsc

N = 10000
E = 160000
F_IN = 128
H = 8
HID = 64
CLS = 16
ND = 160000
ED = 1000000

NDP = 161792
NDH = NDP // 2
NDQ = NDP // 4
NDQS = NDQ // 16
NP = 10112
NPH = NP // 2
NPHS = NPH // 16
NP4 = NP // 4
NP4S = NP4 // 16
EDP = 1003520
EP = 163840

_mesh = plsc.VectorSubcoreMesh(core_axis_name="c", subcore_axis_name="s",
                               num_cores=2, num_subcores=16)

_cp = pltpu.CompilerParams()
_flds = pltpu.CompilerParams.__dataclass_fields__
if "needs_layout_passes" in _flds:
    _cp = dataclasses.replace(_cp, needs_layout_passes=False)
if "use_tc_tiling_on_sc" in _flds:
    _cp = dataclasses.replace(_cp, use_tc_tiling_on_sc=False)

_NEG1 = -1


def _leaky(x):
    return jnp.where(x >= 0, x, 0.2 * x)


def _fill_stripe(buf, shared, row0, nrows, zrows):
    done = 0
    while done < nrows:
        n = min(zrows, nrows - done)
        pltpu.sync_copy(buf.at[pl.ds(0, n)],
                        shared.at[pl.ds(row0 + done, n)])
        done += n


def _dump_stripe(shared, out_hbm, buf, srow0, drow0, nrows, zrows):
    done = 0
    while done < nrows:
        n = min(zrows, nrows - done)
        pltpu.sync_copy(shared.at[pl.ds(srow0 + done, n)],
                        buf.at[pl.ds(0, n)])
        pltpu.sync_copy(buf.at[pl.ds(0, n)],
                        out_hbm.at[pl.ds(drow0 + done, n)])
        done += n


def _mask_pair(sidx, didx, msrc, mdst, lo16, nh16, neg16):
    for o in range(8):
        ob = o * 16
        si = sidx[pl.ds(ob, 16)]
        di = didx[pl.ds(ob, 16)]
        local = di - lo16
        keep = jnp.logical_and(local >= 0, local < nh16)
        msrc[pl.ds(ob, 16)] = jnp.where(keep, si, neg16)
        mdst[pl.ds(ob, 16)] = jnp.where(keep, local, neg16)


def _sc_hist(dstp, ones128, zeros1d):
    @functools.partial(
        pl.kernel,
        out_type=jax.ShapeDtypeStruct((NDP,), jnp.float32),
        mesh=_mesh,
        compiler_params=_cp,
        scratch_types=[
            pltpu.VMEM((128,), jnp.int32),
            pltpu.VMEM((128,), jnp.int32),
            pltpu.VMEM((128,), jnp.float32),
            pltpu.VMEM((512,), jnp.float32),
            pltpu.VMEM_SHARED((NDQ,), jnp.float32),
            pltpu.SemaphoreType.DMA,
        ],
    )
    def k(dst_hbm, ones_hbm, z_hbm, out_hbm, didx, mdst, ones_v, zb, acc,
          sem):
        c = lax.axis_index("c")
        s = lax.axis_index("s")
        nh16 = jnp.full((16,), NDQ, jnp.int32)
        neg16 = jnp.full((16,), _NEG1, jnp.int32)
        pltpu.sync_copy(ones_hbm, ones_v)
        for q in range(2):
            lo = c * NDH + q * NDQ
            lo16 = lax.broadcast(lo, (16,))
            pltpu.sync_copy(z_hbm.at[pl.ds(0, 512)], zb)
            _fill_stripe(zb, acc, s * NDQS, NDQS, 512)
            plsc.subcore_barrier()

            @pl.loop(0, EDP // 16 // 128)
            def _(g):
                base = s * (EDP // 16) + g * 128
                pltpu.sync_copy(dst_hbm.at[pl.ds(base, 128)], didx)
                for o in range(8):
                    ob = o * 16
                    di = didx[pl.ds(ob, 16)]
                    local = di - lo16
                    keep = jnp.logical_and(local >= 0, local < nh16)
                    mdst[pl.ds(ob, 16)] = jnp.where(keep, local, neg16)
                pltpu.sync_copy(
                    ones_v,
                    acc.at[plsc.Indices(mdst, ignored_value=_NEG1)],
                    add=True)

            plsc.subcore_barrier()
            _dump_stripe(acc, out_hbm, zb, s * NDQS, lo + s * NDQS,
                         NDQS, 512)
            plsc.subcore_barrier()

    return k(dstp, ones128, zeros1d)


def _sc_segsum(srcp, dstp, table, zeros2d):
    NSTR = 5
    B = NSTR * 128

    @functools.partial(
        pl.kernel,
        out_type=jax.ShapeDtypeStruct((NDP, 16), jnp.float32),
        mesh=_mesh,
        compiler_params=_cp,
        scratch_types=[
            pltpu.VMEM((B,), jnp.int32),
            pltpu.VMEM((B,), jnp.int32),
            pltpu.VMEM((NSTR, 128), jnp.int32),
            pltpu.VMEM((NSTR, 128), jnp.int32),
            pltpu.VMEM((B, 16), jnp.float32),
            pltpu.VMEM_SHARED((NDQ, 16), jnp.float32),
            pltpu.SemaphoreType.DMA,
            pltpu.SemaphoreType.DMA,
        ],
    )
    def k(src_hbm, dst_hbm, tab, z_hbm, out, sidx, didx, msrc, mdst, rows,
          acc, sem, sem2):
        c = lax.axis_index("c")
        s = lax.axis_index("s")
        nh16 = jnp.full((16,), NDQ, jnp.int32)
        neg16 = jnp.full((16,), _NEG1, jnp.int32)
        for q in range(2):
            lo = c * NDH + q * NDQ
            lo16 = lax.broadcast(lo, (16,))
            pltpu.sync_copy(z_hbm, rows.at[pl.ds(0, 128)])
            _fill_stripe(rows, acc, s * NDQS, NDQS, 128)
            plsc.subcore_barrier()

            @pl.loop(0, EDP // 16 // B)
            def _(g):
                base = s * (EDP // 16) + g * B
                pltpu.sync_copy(src_hbm.at[pl.ds(base, B)], sidx)
                pltpu.sync_copy(dst_hbm.at[pl.ds(base, B)], didx)
                for j in range(NSTR):
                    for o in range(8):
                        ob = j * 128 + o * 16
                        si = sidx[pl.ds(ob, 16)]
                        di = didx[pl.ds(ob, 16)]
                        local = di - lo16
                        keep = jnp.logical_and(local >= 0, local < nh16)
                        msrc[j, pl.ds(o * 16, 16)] = \
                            jnp.where(keep, si, neg16)
                        mdst[j, pl.ds(o * 16, 16)] = \
                            jnp.where(keep, local, neg16)
                gd = [pltpu.async_copy(
                    tab.at[plsc.Indices(msrc.at[j], ignored_value=_NEG1)],
                    rows.at[pl.ds(j * 128, 128)], sem)
                    for j in range(NSTR)]
                for d in gd:
                    d.wait()
                sd = [pltpu.async_copy(
                    rows.at[pl.ds(j * 128, 128)],
                    acc.at[plsc.Indices(mdst.at[j], ignored_value=_NEG1)],
                    sem2, add=True)
                    for j in range(NSTR)]
                for d in sd:
                    d.wait()

            plsc.subcore_barrier()
            _dump_stripe(acc, out, rows, s * NDQS, lo + s * NDQS,
                         NDQS, 128)
            plsc.subcore_barrier()

    return k(srcp, dstp, table, zeros2d)


def _sc_gat1_attn(srcp, dstp, asrcT, adstT, bb, zeros1d):
    @functools.partial(
        pl.kernel,
        out_type=[jax.ShapeDtypeStruct((H * EP,), jnp.float32),
                  jax.ShapeDtypeStruct((2 * NP * 4,), jnp.float32)],
        mesh=_mesh,
        compiler_params=_cp,
        scratch_types=[
            pltpu.VMEM((128,), jnp.int32),
            pltpu.VMEM((128,), jnp.int32),
            pltpu.VMEM((128,), jnp.int32),
            pltpu.VMEM((128,), jnp.float32),
            pltpu.VMEM((512,), jnp.float32),
            pltpu.VMEM((NP,), jnp.float32),
            pltpu.VMEM((NP,), jnp.float32),
            pltpu.VMEM((H, 16), jnp.float32),
            pltpu.VMEM_SHARED((NP * 4,), jnp.float32),
            pltpu.SemaphoreType.DMA,
        ],
    )
    def k(src_hbm, dst_hbm, asrc_hbm, adst_hbm, bb_hbm, z_hbm,
          ex_hbm, s1_hbm, sidx, didx, midx, exbuf, zb, sa, da, bbuf, sacc,
          sem):
        c = lax.axis_index("c")
        s = lax.axis_index("s")
        pltpu.sync_copy(bb_hbm, bbuf)
        pltpu.sync_copy(z_hbm.at[pl.ds(0, 512)], zb)
        _fill_stripe(zb, sacc, s * (NP * 4 // 16), NP * 4 // 16, 512)
        plsc.subcore_barrier()
        for hd in range(H):
            @pl.when(c == hd % 2)
            def _(hd=hd):
                pltpu.sync_copy(asrc_hbm.at[pl.ds(hd * NP, NP)], sa)
                pltpu.sync_copy(adst_hbm.at[pl.ds(hd * NP, NP)], da)
                vb = bbuf[hd, :]

                @pl.loop(0, 80)
                def _(g):
                    base = s * (EP // 16) + g * 128
                    pltpu.sync_copy(src_hbm.at[pl.ds(base, 128)], sidx)
                    pltpu.sync_copy(dst_hbm.at[pl.ds(base, 128)], didx)
                    for o in range(8):
                        ob = o * 16
                        si = sidx[pl.ds(ob, 16)]
                        di = didx[pl.ds(ob, 16)]
                        va = plsc.load_gather(sa, [si])
                        vd = plsc.load_gather(da, [di])
                        ex = jnp.exp(_leaky(va + vd) - vb)
                        exbuf[pl.ds(ob, 16)] = ex
                        midx[pl.ds(ob, 16)] = di * 4 + (hd // 2)
                    pltpu.sync_copy(exbuf, sacc.at[midx], add=True)
                    pltpu.sync_copy(
                        exbuf, ex_hbm.at[pl.ds(hd * EP + base, 128)])

        plsc.subcore_barrier()
        _dump_stripe(sacc, s1_hbm, zb, s * (NP * 4 // 16),
                     c * NP * 4 + s * (NP * 4 // 16), NP * 4 // 16, 512)

    return k(srcp, dstp, asrcT, adstT, bb, zeros1d)


def _sc_gat1_num(srcp, dstp, ex1, hchunks, zeros2d):
    @functools.partial(
        pl.kernel,
        out_type=[jax.ShapeDtypeStruct((NP, 64), jnp.float32)
                  for _ in range(8)],
        mesh=_mesh,
        compiler_params=_cp,
        scratch_types=[
            pltpu.VMEM((128,), jnp.int32),
            pltpu.VMEM((128,), jnp.int32),
            pltpu.VMEM((128,), jnp.int32),
            pltpu.VMEM((128,), jnp.int32),
            pltpu.VMEM((128,), jnp.float32),
            pltpu.VMEM((128, 64), jnp.float32),
            pltpu.VMEM_SHARED((NP4, 64), jnp.float32),
            pltpu.SemaphoreType.DMA,
        ],
    )
    def k(src_hbm, dst_hbm, ex_hbm, h0, h1, h2, h3, h4, h5, h6, h7, z_hbm,
          o0, o1, o2, o3, o4, o5, o6, o7,
          sidx, didx, msrc, mdst, exb, rows, acc, sem):
        hb = [h0, h1, h2, h3, h4, h5, h6, h7]
        outs = [o0, o1, o2, o3, o4, o5, o6, o7]
        c = lax.axis_index("c")
        s = lax.axis_index("s")
        nh16 = jnp.full((16,), NP4, jnp.int32)
        neg16 = jnp.full((16,), _NEG1, jnp.int32)
        iotas = [lax.iota(jnp.int32, 16) + o * 16 for o in range(8)]
        for hd in range(8):
            for q in range(2):
                lo = c * NPH + q * NP4
                lo16 = lax.broadcast(lo, (16,))
                pltpu.sync_copy(z_hbm, rows)
                _fill_stripe(rows, acc, s * NP4S, NP4S, 128)
                plsc.subcore_barrier()

                @pl.loop(0, 80)
                def _(g, hd=hd):
                    base = s * (EP // 16) + g * 128
                    pltpu.sync_copy(src_hbm.at[pl.ds(base, 128)], sidx)
                    pltpu.sync_copy(dst_hbm.at[pl.ds(base, 128)], didx)
                    pltpu.sync_copy(
                        ex_hbm.at[pl.ds(hd * EP + base, 128)], exb)
                    _mask_pair(sidx, didx, msrc, mdst, lo16, nh16, neg16)
                    pltpu.async_copy(
                        hb[hd].at[plsc.Indices(msrc, ignored_value=_NEG1)],
                        rows, sem).wait()

                    @pl.loop(0, 64)
                    def _(f):
                        fs = lax.broadcast(f, (16,))
                        for o in range(8):
                            v = plsc.load_gather(rows, [iotas[o], fs])
                            plsc.store_scatter(rows, [iotas[o], fs],
                                               v * exb[pl.ds(o * 16, 16)])

                    pltpu.sync_copy(
                        rows,
                        acc.at[plsc.Indices(mdst, ignored_value=_NEG1)],
                        add=True)

                plsc.subcore_barrier()
                _dump_stripe(acc, outs[hd], rows, s * NP4S,
                             lo + s * NP4S, NP4S, 128)
                plsc.subcore_barrier()

    return k(srcp, dstp, ex1, *hchunks, zeros2d)


def _sc_gat2(srcp, dstp, asrc2, adst2, b2b, h2g, zeros2d, zeros632):
    @functools.partial(
        pl.kernel,
        out_type=[jax.ShapeDtypeStruct((NP, CLS), jnp.float32),
                  jax.ShapeDtypeStruct((NP,), jnp.float32)],
        mesh=_mesh,
        compiler_params=_cp,
        scratch_types=[
            pltpu.VMEM((128,), jnp.int32),
            pltpu.VMEM((128,), jnp.int32),
            pltpu.VMEM((128,), jnp.int32),
            pltpu.VMEM((128,), jnp.int32),
            pltpu.VMEM((128,), jnp.int32),
            pltpu.VMEM((128,), jnp.float32),
            pltpu.VMEM((632,), jnp.float32),
            pltpu.VMEM((NP,), jnp.float32),
            pltpu.VMEM((NP,), jnp.float32),
            pltpu.VMEM((1, 16), jnp.float32),
            pltpu.VMEM((128, CLS), jnp.float32),
            pltpu.VMEM_SHARED((NP4, CLS), jnp.float32),
            pltpu.VMEM_SHARED((NPH,), jnp.float32),
            pltpu.SemaphoreType.DMA,
        ],
    )
    def k(src_hbm, dst_hbm, sa_hbm, da_hbm, bb_hbm, tab_hbm,
          z_hbm, z632_hbm, num_hbm, s2_hbm,
          sidx, didx, msrc, mdst, ms2, exbuf, zb1, sa, da, bbuf, rows, acc,
          s2sh, sem):
        c = lax.axis_index("c")
        s = lax.axis_index("s")
        hlo16 = lax.broadcast(c * NPH, (16,))
        hnh16 = jnp.full((16,), NPH, jnp.int32)
        nh16 = jnp.full((16,), NP4, jnp.int32)
        neg16 = jnp.full((16,), _NEG1, jnp.int32)
        pltpu.sync_copy(sa_hbm, sa)
        pltpu.sync_copy(da_hbm, da)
        pltpu.sync_copy(bb_hbm, bbuf)
        pltpu.sync_copy(z632_hbm, zb1)
        vb = bbuf[0, :]
        iotas = [lax.iota(jnp.int32, 16) + o * 16 for o in range(8)]

        @pl.when(s < 8)
        def _():
            pltpu.sync_copy(zb1, s2sh.at[pl.ds(s * 632, 632)])

        for q in range(2):
            lo = c * NPH + q * NP4
            lo16 = lax.broadcast(lo, (16,))
            pltpu.sync_copy(z_hbm, rows)
            _fill_stripe(rows, acc, s * NP4S, NP4S, 128)
            plsc.subcore_barrier()

            @pl.loop(0, 80)
            def _(g, q=q):
                base = s * (EP // 16) + g * 128
                pltpu.sync_copy(src_hbm.at[pl.ds(base, 128)], sidx)
                pltpu.sync_copy(dst_hbm.at[pl.ds(base, 128)], didx)
                _mask_pair(sidx, didx, msrc, mdst, lo16, nh16, neg16)
                pltpu.async_copy(
                    tab_hbm.at[plsc.Indices(msrc, ignored_value=_NEG1)],
                    rows, sem).wait()
                for o in range(8):
                    ob = o * 16
                    si = sidx[pl.ds(ob, 16)]
                    di = didx[pl.ds(ob, 16)]
                    va = plsc.load_gather(sa, [si])
                    vd = plsc.load_gather(da, [di])
                    ex = jnp.exp(_leaky(va + vd) - vb)
                    exbuf[pl.ds(ob, 16)] = ex
                    if q == 0:
                        local = di - hlo16
                        keep = jnp.logical_and(local >= 0, local < hnh16)
                        ms2[pl.ds(ob, 16)] = jnp.where(keep, local, neg16)
                if q == 0:
                    pltpu.sync_copy(
                        exbuf,
                        s2sh.at[plsc.Indices(ms2, ignored_value=_NEG1)],
                        add=True)

                @pl.loop(0, CLS)
                def _(f):
                    fs = lax.broadcast(f, (16,))
                    for o in range(8):
                        v = plsc.load_gather(rows, [iotas[o], fs])
                        plsc.store_scatter(rows, [iotas[o], fs],
                                           v * exbuf[pl.ds(o * 16, 16)])

                pltpu.sync_copy(
                    rows, acc.at[plsc.Indices(mdst, ignored_value=_NEG1)],
                    add=True)

            plsc.subcore_barrier()
            _dump_stripe(acc, num_hbm, rows, s * NP4S, lo + s * NP4S,
                         NP4S, 128)
            plsc.subcore_barrier()

        @pl.when(s < 8)
        def _():
            pltpu.sync_copy(s2sh.at[pl.ds(s * 632, 632)], zb1)
            pltpu.sync_copy(zb1,
                            s2_hbm.at[pl.ds(c * NPH + s * 632, 632)])

    return k(srcp, dstp, asrc2, adst2, b2b, h2g, zeros2d, zeros632)


def _t1(dual_x, Wg1, dinv2d):
    BLK = 3200

    def body(x_ref, w_ref, dinv_ref, *g_refs):
        g = dinv_ref[...] * jnp.dot(x_ref[...], w_ref[...],
                                    preferred_element_type=jnp.float32)
        for p in range(4):
            g_refs[p][...] = g[:, 16 * p:16 * p + 16]

    return pl.pallas_call(
        body,
        grid=(ND // BLK,),
        in_specs=[
            pl.BlockSpec((BLK, 4), lambda i: (i, 0)),
            pl.BlockSpec((4, 64), lambda i: (0, 0)),
            pl.BlockSpec((BLK, 1), lambda i: (i, 0)),
        ],
        out_specs=[pl.BlockSpec((BLK, 16), lambda i: (i, 0))
                   for _ in range(4)],
        out_shape=[jax.ShapeDtypeStruct((ND, 16), jnp.float32)
                   for _ in range(4)],
    )(dual_x, Wg1, dinv2d)


def _t2(dual_x, Wg1, Wg2, bg1, dinv2d, s1):
    BLK = 3200

    def body(x_ref, w1_ref, w2_ref, b1_ref, dinv_ref, *rest):
        s_refs = rest[:4]
        g2f_ref = rest[4]
        dinv = dinv_ref[...]
        g1 = dinv * jnp.dot(x_ref[...], w1_ref[...],
                            preferred_element_type=jnp.float32)
        S = jnp.concatenate([s_refs[p][...] for p in range(4)], axis=1)
        q1 = dinv * (S + g1) + b1_ref[0, :]
        h2 = jnp.dot(jnp.maximum(q1, 0.0), w2_ref[...],
                     preferred_element_type=jnp.float32)
        g2f_ref[...] = dinv * h2

    return pl.pallas_call(
        body,
        grid=(ND // BLK,),
        in_specs=[
            pl.BlockSpec((BLK, 4), lambda i: (i, 0)),
            pl.BlockSpec((4, 64), lambda i: (0, 0)),
            pl.BlockSpec((64, 16), lambda i: (0, 0)),
            pl.BlockSpec((1, 64), lambda i: (0, 0)),
            pl.BlockSpec((BLK, 1), lambda i: (i, 0)),
        ] + [pl.BlockSpec((BLK, 16), lambda i: (i, 0)) for _ in range(4)],
        out_specs=pl.BlockSpec((BLK, 16), lambda i: (i, 0)),
        out_shape=jax.ShapeDtypeStruct((ND, 16), jnp.float32),
    )(dual_x, Wg1, Wg2, bg1.reshape(1, 64), dinv2d, *s1)


def _t3(dinv2d, s2, g2f, bg2):
    BLK = 3200

    def body(dinv_ref, s_ref, g_ref, b_ref, out_ref):
        out_ref[...] = (dinv_ref[...] * (s_ref[...] + g_ref[...])
                        + b_ref[0, :])

    return pl.pallas_call(
        body,
        grid=(ND // BLK,),
        in_specs=[
            pl.BlockSpec((BLK, 1), lambda i: (i, 0)),
            pl.BlockSpec((BLK, 16), lambda i: (i, 0)),
            pl.BlockSpec((BLK, 16), lambda i: (i, 0)),
            pl.BlockSpec((1, 16), lambda i: (0, 0)),
        ],
        out_specs=pl.BlockSpec((BLK, 16), lambda i: (i, 0)),
        out_shape=jax.ShapeDtypeStruct((ND, 16), jnp.float32),
    )(dinv2d, s2, g2f, bg2.reshape(1, 16))


def _t4(x, W1, att_src1, att_dst1):
    BLK = 1000

    def body(*refs):
        x_ref, w_ref, as_ref, ad_ref = refs[:4]
        hrefs = refs[4:12]
        at_ref, dt_ref, ms_ref, md_ref = refs[12:]
        i = pl.program_id(0)
        h = jnp.dot(x_ref[...], w_ref[...],
                    preferred_element_type=jnp.float32)

        @pl.when(i == 0)
        def _():
            ms_ref[...] = jnp.full((H, 128), -jnp.inf, jnp.float32)
            md_ref[...] = jnp.full((H, 128), -jnp.inf, jnp.float32)

        for hd in range(H):
            hcol = h[:, 64 * hd:64 * hd + 64]
            hrefs[hd][...] = hcol
            va = jnp.dot(hcol, as_ref[hd, :],
                         preferred_element_type=jnp.float32)
            vd = jnp.dot(hcol, ad_ref[hd, :],
                         preferred_element_type=jnp.float32)
            at_ref[:, hd:hd + 1] = va[:, None]
            dt_ref[:, hd:hd + 1] = vd[:, None]
            ms_ref[hd, :] = jnp.maximum(ms_ref[hd, :], jnp.max(va))
            md_ref[hd, :] = jnp.maximum(md_ref[hd, :], jnp.max(vd))

    return pl.pallas_call(
        body,
        grid=(N // BLK,),
        in_specs=[
            pl.BlockSpec((BLK, F_IN), lambda i: (i, 0)),
            pl.BlockSpec((F_IN, H * HID), lambda i: (0, 0)),
            pl.BlockSpec((H, HID), lambda i: (0, 0)),
            pl.BlockSpec((H, HID), lambda i: (0, 0)),
        ],
        out_specs=[pl.BlockSpec((BLK, 64), lambda i: (i, 0))
                   for _ in range(8)] +
                  [pl.BlockSpec((BLK, H), lambda i: (i, 0)),
                   pl.BlockSpec((BLK, H), lambda i: (i, 0)),
                   pl.BlockSpec((H, 128), lambda i: (0, 0)),
                   pl.BlockSpec((H, 128), lambda i: (0, 0))],
        out_shape=[jax.ShapeDtypeStruct((N, 64), jnp.float32)
                   for _ in range(8)] +
                  [jax.ShapeDtypeStruct((N, H), jnp.float32),
                   jax.ShapeDtypeStruct((N, H), jnp.float32),
                   jax.ShapeDtypeStruct((H, 128), jnp.float32),
                   jax.ShapeDtypeStruct((H, 128), jnp.float32)],
    )(x, W1, att_src1, att_dst1)


def _t5(nums, hs, s1parts, asrc, adst, bb, b1, W2, att_src2, att_dst2):
    BLK = 1000

    def body(*refs):
        nrefs = refs[:8]
        hrefs = refs[8:16]
        (s1_ref, at_ref, dt_ref, bb_ref, b1_ref, w2_ref, as2_ref, ad2_ref,
         hg_ref, a2_ref, d2_ref, m2s_ref, m2d_ref) = refs[16:]
        i = pl.program_id(0)
        cols = []
        for hd in range(H):
            exs = jnp.exp(_leaky(at_ref[:, hd:hd + 1] + dt_ref[:, hd:hd + 1])
                          - bb_ref[hd, 0])
            den = (s1_ref[hd % 2, :, hd // 2:hd // 2 + 1] + exs + 1e-16)
            cols.append((nrefs[hd][...] + exs * hrefs[hd][...]) / den)
        out1 = jnp.concatenate(cols, axis=1) + b1_ref[0, :]
        hh = jnp.where(out1 > 0, out1, jnp.exp(jnp.minimum(out1, 0.0)) - 1.0)
        hg = jnp.dot(hh, w2_ref[...], preferred_element_type=jnp.float32)
        a2 = jnp.dot(hg, as2_ref[0, :], preferred_element_type=jnp.float32)
        d2 = jnp.dot(hg, ad2_ref[0, :], preferred_element_type=jnp.float32)
        hg_ref[...] = hg
        a2_ref[...] = a2[:, None]
        d2_ref[...] = d2[:, None]

        @pl.when(i == 0)
        def _():
            m2s_ref[...] = jnp.full((8, 128), -jnp.inf, jnp.float32)
            md = jnp.full((8, 128), -jnp.inf, jnp.float32)
            m2d_ref[...] = md

        m2s_ref[...] = jnp.maximum(m2s_ref[...], jnp.max(a2))
        m2d_ref[...] = jnp.maximum(m2d_ref[...], jnp.max(d2))

    return pl.pallas_call(
        body,
        grid=(N // BLK,),
        in_specs=[pl.BlockSpec((BLK, 64), lambda i: (i, 0))
                  for _ in range(8)] +
                 [pl.BlockSpec((BLK, 64), lambda i: (i, 0))
                  for _ in range(8)] +
                 [pl.BlockSpec((2, BLK, 4), lambda i: (0, i, 0)),
                  pl.BlockSpec((BLK, H), lambda i: (i, 0)),
                  pl.BlockSpec((BLK, H), lambda i: (i, 0)),
                  pl.BlockSpec((H, 16), lambda i: (0, 0)),
                  pl.BlockSpec((1, 512), lambda i: (0, 0)),
                  pl.BlockSpec((512, CLS), lambda i: (0, 0)),
                  pl.BlockSpec((1, CLS), lambda i: (0, 0)),
                  pl.BlockSpec((1, CLS), lambda i: (0, 0))],
        out_specs=[pl.BlockSpec((BLK, CLS), lambda i: (i, 0)),
                   pl.BlockSpec((BLK, 1), lambda i: (i, 0)),
                   pl.BlockSpec((BLK, 1), lambda i: (i, 0)),
                   pl.BlockSpec((8, 128), lambda i: (0, 0)),
                   pl.BlockSpec((8, 128), lambda i: (0, 0))],
        out_shape=[jax.ShapeDtypeStruct((N, CLS), jnp.float32),
                   jax.ShapeDtypeStruct((N, 1), jnp.float32),
                   jax.ShapeDtypeStruct((N, 1), jnp.float32),
                   jax.ShapeDtypeStruct((8, 128), jnp.float32),
                   jax.ShapeDtypeStruct((8, 128), jnp.float32)],
    )(*nums, *hs, s1parts, asrc, adst, bb, b1.reshape(1, 512), W2,
      att_src2, att_dst2)


def _t6(num2, s2, h2g, asrc2, adst2, b2b, b2):
    BLK = 2000

    def body(n_ref, s_ref, hg_ref, a2_ref, d2_ref, bb_ref, b2_ref, out_ref):
        exs = jnp.exp(_leaky(a2_ref[...] + d2_ref[...]) - bb_ref[0, 0])
        den = s_ref[...] + exs + 1e-16
        out_ref[...] = (n_ref[...] + exs * hg_ref[...]) / den + b2_ref[0, :]

    return pl.pallas_call(
        body,
        grid=(N // BLK,),
        in_specs=[
            pl.BlockSpec((BLK, CLS), lambda i: (i, 0)),
            pl.BlockSpec((BLK, 1), lambda i: (i, 0)),
            pl.BlockSpec((BLK, CLS), lambda i: (i, 0)),
            pl.BlockSpec((BLK, 1), lambda i: (i, 0)),
            pl.BlockSpec((BLK, 1), lambda i: (i, 0)),
            pl.BlockSpec((1, 16), lambda i: (0, 0)),
            pl.BlockSpec((1, CLS), lambda i: (0, 0)),
        ],
        out_specs=pl.BlockSpec((BLK, CLS), lambda i: (i, 0)),
        out_shape=jax.ShapeDtypeStruct((N, CLS), jnp.float32),
    )(num2, s2, h2g, asrc2, adst2, b2b, b2.reshape(1, CLS))


def kernel(x, edge_index, dual_x, dual_edge_index, epoch,
           W1, att_src1, att_dst1, b1, W2, att_src2, att_dst2, b2,
           Wg1, bg1, Wg2, bg2):
    f32 = jnp.float32
    i32 = jnp.int32

    dsrc = jnp.concatenate(
        [dual_edge_index[0], jnp.zeros((EDP - ED,), i32)])
    ddst = jnp.concatenate(
        [dual_edge_index[1], jnp.full((EDP - ED,), ND, i32)])
    psrc = jnp.concatenate([edge_index[0], jnp.zeros((EP - E,), i32)])
    pdst = jnp.concatenate([edge_index[1], jnp.full((EP - E,), N, i32)])

    ones128 = jnp.ones((128,), f32)
    z1d = jnp.zeros((2048,), f32)
    z2d16 = jnp.zeros((128, 16), f32)
    z2d64 = jnp.zeros((128, 64), f32)
    z632 = jnp.zeros((632,), f32)

    histp = _sc_hist(ddst, ones128, z1d)
    deg = histp[:ND] + 1.0
    dinv2d = lax.rsqrt(deg)[:, None]
    g1 = _t1(dual_x, Wg1, dinv2d)
    s1 = [_sc_segsum(dsrc, ddst, g1[j], z2d16)[:ND] for j in range(4)]
    g2f = _t2(dual_x, Wg1, Wg2, bg1, dinv2d, s1)
    s2 = _sc_segsum(dsrc, ddst, g2f, z2d16)[:ND]
    Q = _t3(dinv2d, s2, g2f, bg2)

    t4 = _t4(x, W1, att_src1, att_dst1)
    hs = list(t4[0:8])
    asrc, adst, msrc, mdst = t4[8], t4[9], t4[10], t4[11]
    B = _leaky(jnp.max(msrc, axis=1) + jnp.max(mdst, axis=1))
    bb = jnp.broadcast_to(B[:, None], (H, 16)).astype(f32)
    asrcTp = jnp.pad(asrc.T, ((0, 0), (0, NP - N))).reshape(H * NP)
    adstTp = jnp.pad(adst.T, ((0, 0), (0, NP - N))).reshape(H * NP)
    ex1, s1flat = _sc_gat1_attn(psrc, pdst, asrcTp, adstTp, bb, z1d)
    s1parts = s1flat.reshape(2, NP, 4)[:, :N, :]
    nums = _sc_gat1_num(psrc, pdst, ex1, hs, z2d64)
    nums = [a[:N] for a in nums]
    h2g, asrc2, adst2, m2s, m2d = _t5(
        nums, hs, s1parts, asrc, adst, bb, b1, W2, att_src2, att_dst2)
    B2 = _leaky(jnp.max(m2s) + jnp.max(m2d))
    b2b = jnp.broadcast_to(B2, (1, 16)).astype(f32)
    a2p = jnp.pad(asrc2.reshape(N), (0, NP - N))
    d2p = jnp.pad(adst2.reshape(N), (0, NP - N))
    num2, s2g = _sc_gat2(psrc, pdst, a2p, d2p, b2b, h2g, z2d16, z632)
    out = _t6(num2[:N], s2g[:N].reshape(N, 1), h2g, asrc2, adst2, b2b, b2)
    return (out, Q)


# speedup vs baseline: 2.8709x; 1.0513x over previous
import dataclasses
import functools

import jax
import jax.numpy as jnp
from jax import lax
from jax.experimental import pallas as pl
from jax.experimental.pallas import tpu as pltpu
from jax.experimental.pallas import tpu_sc as plsc

N = 10000
E = 160000
F_IN = 128
H = 8
HID = 64
CLS = 16
ND = 160000
ED = 1000000

NDP = 161792
NDH = NDP // 2
NDQ = NDP // 4
NDQS = NDQ // 16
NP = 10112
NPH = NP // 2
NPHS = NPH // 16
NP4 = NP // 4
NP4S = NP4 // 16
EDP = 1003520
EP = 163840

_mesh = plsc.VectorSubcoreMesh(core_axis_name="c", subcore_axis_name="s",
                               num_cores=2, num_subcores=16)

_cp = pltpu.CompilerParams()
_flds = pltpu.CompilerParams.__dataclass_fields__
if "needs_layout_passes" in _flds:
    _cp = dataclasses.replace(_cp, needs_layout_passes=False)
if "use_tc_tiling_on_sc" in _flds:
    _cp = dataclasses.replace(_cp, use_tc_tiling_on_sc=False)

_NEG1 = -1


def _leaky(x):
    return jnp.where(x >= 0, x, 0.2 * x)


def _fill_stripe(buf, shared, row0, nrows, zrows):
    done = 0
    while done < nrows:
        n = min(zrows, nrows - done)
        pltpu.sync_copy(buf.at[pl.ds(0, n)],
                        shared.at[pl.ds(row0 + done, n)])
        done += n


def _dump_stripe(shared, out_hbm, buf, srow0, drow0, nrows, zrows):
    done = 0
    while done < nrows:
        n = min(zrows, nrows - done)
        pltpu.sync_copy(shared.at[pl.ds(srow0 + done, n)],
                        buf.at[pl.ds(0, n)])
        pltpu.sync_copy(buf.at[pl.ds(0, n)],
                        out_hbm.at[pl.ds(drow0 + done, n)])
        done += n


def _mask_pair(sidx, didx, msrc, mdst, lo16, nh16, neg16):
    for o in range(8):
        ob = o * 16
        si = sidx[pl.ds(ob, 16)]
        di = didx[pl.ds(ob, 16)]
        local = di - lo16
        keep = jnp.logical_and(local >= 0, local < nh16)
        msrc[pl.ds(ob, 16)] = jnp.where(keep, si, neg16)
        mdst[pl.ds(ob, 16)] = jnp.where(keep, local, neg16)


def _sc_hist(dstp, ones128, zeros1d):
    @functools.partial(
        pl.kernel,
        out_type=jax.ShapeDtypeStruct((NDP,), jnp.float32),
        mesh=_mesh,
        compiler_params=_cp,
        scratch_types=[
            pltpu.VMEM((128,), jnp.int32),
            pltpu.VMEM((128,), jnp.int32),
            pltpu.VMEM((128,), jnp.float32),
            pltpu.VMEM((512,), jnp.float32),
            pltpu.VMEM_SHARED((NDQ,), jnp.float32),
            pltpu.SemaphoreType.DMA,
        ],
    )
    def k(dst_hbm, ones_hbm, z_hbm, out_hbm, didx, mdst, ones_v, zb, acc,
          sem):
        c = lax.axis_index("c")
        s = lax.axis_index("s")
        nh16 = jnp.full((16,), NDQ, jnp.int32)
        neg16 = jnp.full((16,), _NEG1, jnp.int32)
        pltpu.sync_copy(ones_hbm, ones_v)
        for q in range(2):
            lo = c * NDH + q * NDQ
            lo16 = lax.broadcast(lo, (16,))
            pltpu.sync_copy(z_hbm.at[pl.ds(0, 512)], zb)
            _fill_stripe(zb, acc, s * NDQS, NDQS, 512)
            plsc.subcore_barrier()

            @pl.loop(0, EDP // 16 // 128)
            def _(g):
                base = s * (EDP // 16) + g * 128
                pltpu.sync_copy(dst_hbm.at[pl.ds(base, 128)], didx)
                for o in range(8):
                    ob = o * 16
                    di = didx[pl.ds(ob, 16)]
                    local = di - lo16
                    keep = jnp.logical_and(local >= 0, local < nh16)
                    mdst[pl.ds(ob, 16)] = jnp.where(keep, local, neg16)
                pltpu.sync_copy(
                    ones_v,
                    acc.at[plsc.Indices(mdst, ignored_value=_NEG1)],
                    add=True)

            plsc.subcore_barrier()
            _dump_stripe(acc, out_hbm, zb, s * NDQS, lo + s * NDQS,
                         NDQS, 512)
            plsc.subcore_barrier()

    return k(dstp, ones128, zeros1d)


def _sc_segsum(srcp, dstp, table, zeros2d):
    NSTR = 5
    B = NSTR * 128

    @functools.partial(
        pl.kernel,
        out_type=jax.ShapeDtypeStruct((NDP, 16), jnp.float32),
        mesh=_mesh,
        compiler_params=_cp,
        scratch_types=[
            pltpu.VMEM((B,), jnp.int32),
            pltpu.VMEM((B,), jnp.int32),
            pltpu.VMEM((NSTR, 128), jnp.int32),
            pltpu.VMEM((NSTR, 128), jnp.int32),
            pltpu.VMEM((B, 16), jnp.float32),
            pltpu.VMEM_SHARED((NDQ, 16), jnp.float32),
            pltpu.SemaphoreType.DMA,
            pltpu.SemaphoreType.DMA,
        ],
    )
    def k(src_hbm, dst_hbm, tab, z_hbm, out, sidx, didx, msrc, mdst, rows,
          acc, sem, sem2):
        c = lax.axis_index("c")
        s = lax.axis_index("s")
        nh16 = jnp.full((16,), NDQ, jnp.int32)
        neg16 = jnp.full((16,), _NEG1, jnp.int32)
        for q in range(2):
            lo = c * NDH + q * NDQ
            lo16 = lax.broadcast(lo, (16,))
            pltpu.sync_copy(z_hbm, rows.at[pl.ds(0, 128)])
            _fill_stripe(rows, acc, s * NDQS, NDQS, 128)
            plsc.subcore_barrier()

            @pl.loop(0, EDP // 16 // B)
            def _(g):
                base = s * (EDP // 16) + g * B
                pltpu.sync_copy(src_hbm.at[pl.ds(base, B)], sidx)
                pltpu.sync_copy(dst_hbm.at[pl.ds(base, B)], didx)
                for j in range(NSTR):
                    for o in range(8):
                        ob = j * 128 + o * 16
                        si = sidx[pl.ds(ob, 16)]
                        di = didx[pl.ds(ob, 16)]
                        local = di - lo16
                        keep = jnp.logical_and(local >= 0, local < nh16)
                        msrc[j, pl.ds(o * 16, 16)] = \
                            jnp.where(keep, si, neg16)
                        mdst[j, pl.ds(o * 16, 16)] = \
                            jnp.where(keep, local, neg16)
                gd = [pltpu.async_copy(
                    tab.at[plsc.Indices(msrc.at[j], ignored_value=_NEG1)],
                    rows.at[pl.ds(j * 128, 128)], sem)
                    for j in range(NSTR)]
                for d in gd:
                    d.wait()
                sd = [pltpu.async_copy(
                    rows.at[pl.ds(j * 128, 128)],
                    acc.at[plsc.Indices(mdst.at[j], ignored_value=_NEG1)],
                    sem2, add=True)
                    for j in range(NSTR)]
                for d in sd:
                    d.wait()

            plsc.subcore_barrier()
            _dump_stripe(acc, out, rows, s * NDQS, lo + s * NDQS,
                         NDQS, 128)
            plsc.subcore_barrier()

    return k(srcp, dstp, table, zeros2d)


def _sc_gat1_attn(srcp, dstp, asrcT, adstT, bb, zeros1d):
    @functools.partial(
        pl.kernel,
        out_type=[jax.ShapeDtypeStruct((H * EP,), jnp.float32),
                  jax.ShapeDtypeStruct((2 * NP * 4,), jnp.float32)],
        mesh=_mesh,
        compiler_params=_cp,
        scratch_types=[
            pltpu.VMEM((128,), jnp.int32),
            pltpu.VMEM((128,), jnp.int32),
            pltpu.VMEM((128,), jnp.int32),
            pltpu.VMEM((128,), jnp.float32),
            pltpu.VMEM((512,), jnp.float32),
            pltpu.VMEM((NP,), jnp.float32),
            pltpu.VMEM((NP,), jnp.float32),
            pltpu.VMEM((H, 16), jnp.float32),
            pltpu.VMEM_SHARED((NP * 4,), jnp.float32),
            pltpu.SemaphoreType.DMA,
        ],
    )
    def k(src_hbm, dst_hbm, asrc_hbm, adst_hbm, bb_hbm, z_hbm,
          ex_hbm, s1_hbm, sidx, didx, midx, exbuf, zb, sa, da, bbuf, sacc,
          sem):
        c = lax.axis_index("c")
        s = lax.axis_index("s")
        pltpu.sync_copy(bb_hbm, bbuf)
        pltpu.sync_copy(z_hbm.at[pl.ds(0, 512)], zb)
        _fill_stripe(zb, sacc, s * (NP * 4 // 16), NP * 4 // 16, 512)
        plsc.subcore_barrier()
        for hd in range(H):
            @pl.when(c == hd % 2)
            def _(hd=hd):
                pltpu.sync_copy(asrc_hbm.at[pl.ds(hd * NP, NP)], sa)
                pltpu.sync_copy(adst_hbm.at[pl.ds(hd * NP, NP)], da)
                vb = bbuf[hd, :]

                @pl.loop(0, 80)
                def _(g):
                    base = s * (EP // 16) + g * 128
                    pltpu.sync_copy(src_hbm.at[pl.ds(base, 128)], sidx)
                    pltpu.sync_copy(dst_hbm.at[pl.ds(base, 128)], didx)
                    for o in range(8):
                        ob = o * 16
                        si = sidx[pl.ds(ob, 16)]
                        di = didx[pl.ds(ob, 16)]
                        va = plsc.load_gather(sa, [si])
                        vd = plsc.load_gather(da, [di])
                        ex = jnp.exp(_leaky(va + vd) - vb)
                        exbuf[pl.ds(ob, 16)] = ex
                        midx[pl.ds(ob, 16)] = di * 4 + (hd // 2)
                    pltpu.sync_copy(exbuf, sacc.at[midx], add=True)
                    pltpu.sync_copy(
                        exbuf, ex_hbm.at[pl.ds(hd * EP + base, 128)])

        plsc.subcore_barrier()
        _dump_stripe(sacc, s1_hbm, zb, s * (NP * 4 // 16),
                     c * NP * 4 + s * (NP * 4 // 16), NP * 4 // 16, 512)

    return k(srcp, dstp, asrcT, adstT, bb, zeros1d)


def _sc_gat1_num(srcp, dstp, ex1, hchunks, zeros2d):
    @functools.partial(
        pl.kernel,
        out_type=[jax.ShapeDtypeStruct((NP, 64), jnp.float32)
                  for _ in range(8)],
        mesh=_mesh,
        compiler_params=_cp,
        scratch_types=[
            pltpu.VMEM((128,), jnp.int32),
            pltpu.VMEM((128,), jnp.int32),
            pltpu.VMEM((128,), jnp.int32),
            pltpu.VMEM((128,), jnp.int32),
            pltpu.VMEM((128,), jnp.float32),
            pltpu.VMEM((128, 64), jnp.float32),
            pltpu.VMEM_SHARED((NP4, 64), jnp.float32),
            pltpu.SemaphoreType.DMA,
            pltpu.SemaphoreType.DMA,
        ],
    )
    def k(src_hbm, dst_hbm, ex_hbm, h0, h1, h2, h3, h4, h5, h6, h7, z_hbm,
          o0, o1, o2, o3, o4, o5, o6, o7,
          sidx, didx, msrc, mdst, exb, rows, acc, sem, sem2):
        hb = [h0, h1, h2, h3, h4, h5, h6, h7]
        outs = [o0, o1, o2, o3, o4, o5, o6, o7]
        c = lax.axis_index("c")
        s = lax.axis_index("s")
        nh16 = jnp.full((16,), NP4, jnp.int32)
        neg16 = jnp.full((16,), _NEG1, jnp.int32)
        iotas = [lax.iota(jnp.int32, 16) + o * 16 for o in range(8)]
        for hd in range(8):
            for q in range(2):
                lo = c * NPH + q * NP4
                lo16 = lax.broadcast(lo, (16,))
                pltpu.sync_copy(z_hbm, rows)
                _fill_stripe(rows, acc, s * NP4S, NP4S, 128)
                plsc.subcore_barrier()

                @pl.loop(0, 80)
                def _(g, hd=hd):
                    base = s * (EP // 16) + g * 128
                    ld = [pltpu.async_copy(
                              src_hbm.at[pl.ds(base, 128)], sidx, sem2),
                          pltpu.async_copy(
                              dst_hbm.at[pl.ds(base, 128)], didx, sem2),
                          pltpu.async_copy(
                              ex_hbm.at[pl.ds(hd * EP + base, 128)], exb,
                              sem2)]
                    for d in ld:
                        d.wait()
                    _mask_pair(sidx, didx, msrc, mdst, lo16, nh16, neg16)
                    pltpu.async_copy(
                        hb[hd].at[plsc.Indices(msrc, ignored_value=_NEG1)],
                        rows, sem).wait()

                    @pl.loop(0, 64, step=8)
                    def _(f0):
                        for kk in range(8):
                            fs = lax.broadcast(f0 + kk, (16,))
                            for o in range(8):
                                v = plsc.load_gather(rows, [iotas[o], fs])
                                plsc.store_scatter(
                                    rows, [iotas[o], fs],
                                    v * exb[pl.ds(o * 16, 16)])

                    pltpu.sync_copy(
                        rows,
                        acc.at[plsc.Indices(mdst, ignored_value=_NEG1)],
                        add=True)

                plsc.subcore_barrier()
                _dump_stripe(acc, outs[hd], rows, s * NP4S,
                             lo + s * NP4S, NP4S, 128)
                plsc.subcore_barrier()

    return k(srcp, dstp, ex1, *hchunks, zeros2d)


def _sc_gat2(srcp, dstp, asrc2, adst2, b2b, h2g, zeros2d, zeros632):
    @functools.partial(
        pl.kernel,
        out_type=[jax.ShapeDtypeStruct((NP, CLS), jnp.float32),
                  jax.ShapeDtypeStruct((NP,), jnp.float32)],
        mesh=_mesh,
        compiler_params=_cp,
        scratch_types=[
            pltpu.VMEM((128,), jnp.int32),
            pltpu.VMEM((128,), jnp.int32),
            pltpu.VMEM((128,), jnp.int32),
            pltpu.VMEM((128,), jnp.int32),
            pltpu.VMEM((128,), jnp.int32),
            pltpu.VMEM((128,), jnp.float32),
            pltpu.VMEM((632,), jnp.float32),
            pltpu.VMEM((NP,), jnp.float32),
            pltpu.VMEM((NP,), jnp.float32),
            pltpu.VMEM((1, 16), jnp.float32),
            pltpu.VMEM((128, CLS), jnp.float32),
            pltpu.VMEM_SHARED((NP4, CLS), jnp.float32),
            pltpu.VMEM_SHARED((NPH,), jnp.float32),
            pltpu.SemaphoreType.DMA,
        ],
    )
    def k(src_hbm, dst_hbm, sa_hbm, da_hbm, bb_hbm, tab_hbm,
          z_hbm, z632_hbm, num_hbm, s2_hbm,
          sidx, didx, msrc, mdst, ms2, exbuf, zb1, sa, da, bbuf, rows, acc,
          s2sh, sem):
        c = lax.axis_index("c")
        s = lax.axis_index("s")
        hlo16 = lax.broadcast(c * NPH, (16,))
        hnh16 = jnp.full((16,), NPH, jnp.int32)
        nh16 = jnp.full((16,), NP4, jnp.int32)
        neg16 = jnp.full((16,), _NEG1, jnp.int32)
        pltpu.sync_copy(sa_hbm, sa)
        pltpu.sync_copy(da_hbm, da)
        pltpu.sync_copy(bb_hbm, bbuf)
        pltpu.sync_copy(z632_hbm, zb1)
        vb = bbuf[0, :]
        iotas = [lax.iota(jnp.int32, 16) + o * 16 for o in range(8)]

        @pl.when(s < 8)
        def _():
            pltpu.sync_copy(zb1, s2sh.at[pl.ds(s * 632, 632)])

        for q in range(2):
            lo = c * NPH + q * NP4
            lo16 = lax.broadcast(lo, (16,))
            pltpu.sync_copy(z_hbm, rows)
            _fill_stripe(rows, acc, s * NP4S, NP4S, 128)
            plsc.subcore_barrier()

            @pl.loop(0, 80)
            def _(g, q=q):
                base = s * (EP // 16) + g * 128
                pltpu.sync_copy(src_hbm.at[pl.ds(base, 128)], sidx)
                pltpu.sync_copy(dst_hbm.at[pl.ds(base, 128)], didx)
                _mask_pair(sidx, didx, msrc, mdst, lo16, nh16, neg16)
                pltpu.async_copy(
                    tab_hbm.at[plsc.Indices(msrc, ignored_value=_NEG1)],
                    rows, sem).wait()
                for o in range(8):
                    ob = o * 16
                    si = sidx[pl.ds(ob, 16)]
                    di = didx[pl.ds(ob, 16)]
                    va = plsc.load_gather(sa, [si])
                    vd = plsc.load_gather(da, [di])
                    ex = jnp.exp(_leaky(va + vd) - vb)
                    exbuf[pl.ds(ob, 16)] = ex
                    if q == 0:
                        local = di - hlo16
                        keep = jnp.logical_and(local >= 0, local < hnh16)
                        ms2[pl.ds(ob, 16)] = jnp.where(keep, local, neg16)
                if q == 0:
                    pltpu.sync_copy(
                        exbuf,
                        s2sh.at[plsc.Indices(ms2, ignored_value=_NEG1)],
                        add=True)

                @pl.loop(0, CLS, step=8)
                def _(f0):
                    for kk in range(8):
                        fs = lax.broadcast(f0 + kk, (16,))
                        for o in range(8):
                            v = plsc.load_gather(rows, [iotas[o], fs])
                            plsc.store_scatter(
                                rows, [iotas[o], fs],
                                v * exbuf[pl.ds(o * 16, 16)])

                pltpu.sync_copy(
                    rows, acc.at[plsc.Indices(mdst, ignored_value=_NEG1)],
                    add=True)

            plsc.subcore_barrier()
            _dump_stripe(acc, num_hbm, rows, s * NP4S, lo + s * NP4S,
                         NP4S, 128)
            plsc.subcore_barrier()

        @pl.when(s < 8)
        def _():
            pltpu.sync_copy(s2sh.at[pl.ds(s * 632, 632)], zb1)
            pltpu.sync_copy(zb1,
                            s2_hbm.at[pl.ds(c * NPH + s * 632, 632)])

    return k(srcp, dstp, asrc2, adst2, b2b, h2g, zeros2d, zeros632)


def _t1(dual_x, Wg1, dinv2d):
    BLK = 3200

    def body(x_ref, w_ref, dinv_ref, *g_refs):
        g = dinv_ref[...] * jnp.dot(x_ref[...], w_ref[...],
                                    preferred_element_type=jnp.float32)
        for p in range(4):
            g_refs[p][...] = g[:, 16 * p:16 * p + 16]

    return pl.pallas_call(
        body,
        grid=(ND // BLK,),
        in_specs=[
            pl.BlockSpec((BLK, 4), lambda i: (i, 0)),
            pl.BlockSpec((4, 64), lambda i: (0, 0)),
            pl.BlockSpec((BLK, 1), lambda i: (i, 0)),
        ],
        out_specs=[pl.BlockSpec((BLK, 16), lambda i: (i, 0))
                   for _ in range(4)],
        out_shape=[jax.ShapeDtypeStruct((ND, 16), jnp.float32)
                   for _ in range(4)],
    )(dual_x, Wg1, dinv2d)


def _t2(dual_x, Wg1, Wg2, bg1, dinv2d, s1):
    BLK = 3200

    def body(x_ref, w1_ref, w2_ref, b1_ref, dinv_ref, *rest):
        s_refs = rest[:4]
        g2f_ref = rest[4]
        dinv = dinv_ref[...]
        g1 = dinv * jnp.dot(x_ref[...], w1_ref[...],
                            preferred_element_type=jnp.float32)
        S = jnp.concatenate([s_refs[p][...] for p in range(4)], axis=1)
        q1 = dinv * (S + g1) + b1_ref[0, :]
        h2 = jnp.dot(jnp.maximum(q1, 0.0), w2_ref[...],
                     preferred_element_type=jnp.float32)
        g2f_ref[...] = dinv * h2

    return pl.pallas_call(
        body,
        grid=(ND // BLK,),
        in_specs=[
            pl.BlockSpec((BLK, 4), lambda i: (i, 0)),
            pl.BlockSpec((4, 64), lambda i: (0, 0)),
            pl.BlockSpec((64, 16), lambda i: (0, 0)),
            pl.BlockSpec((1, 64), lambda i: (0, 0)),
            pl.BlockSpec((BLK, 1), lambda i: (i, 0)),
        ] + [pl.BlockSpec((BLK, 16), lambda i: (i, 0)) for _ in range(4)],
        out_specs=pl.BlockSpec((BLK, 16), lambda i: (i, 0)),
        out_shape=jax.ShapeDtypeStruct((ND, 16), jnp.float32),
    )(dual_x, Wg1, Wg2, bg1.reshape(1, 64), dinv2d, *s1)


def _t3(dinv2d, s2, g2f, bg2):
    BLK = 3200

    def body(dinv_ref, s_ref, g_ref, b_ref, out_ref):
        out_ref[...] = (dinv_ref[...] * (s_ref[...] + g_ref[...])
                        + b_ref[0, :])

    return pl.pallas_call(
        body,
        grid=(ND // BLK,),
        in_specs=[
            pl.BlockSpec((BLK, 1), lambda i: (i, 0)),
            pl.BlockSpec((BLK, 16), lambda i: (i, 0)),
            pl.BlockSpec((BLK, 16), lambda i: (i, 0)),
            pl.BlockSpec((1, 16), lambda i: (0, 0)),
        ],
        out_specs=pl.BlockSpec((BLK, 16), lambda i: (i, 0)),
        out_shape=jax.ShapeDtypeStruct((ND, 16), jnp.float32),
    )(dinv2d, s2, g2f, bg2.reshape(1, 16))


def _t4(x, W1, att_src1, att_dst1):
    BLK = 1000

    def body(*refs):
        x_ref, w_ref, as_ref, ad_ref = refs[:4]
        hrefs = refs[4:12]
        at_ref, dt_ref, ms_ref, md_ref = refs[12:]
        i = pl.program_id(0)
        h = jnp.dot(x_ref[...], w_ref[...],
                    preferred_element_type=jnp.float32)

        @pl.when(i == 0)
        def _():
            ms_ref[...] = jnp.full((H, 128), -jnp.inf, jnp.float32)
            md_ref[...] = jnp.full((H, 128), -jnp.inf, jnp.float32)

        for hd in range(H):
            hcol = h[:, 64 * hd:64 * hd + 64]
            hrefs[hd][...] = hcol
            va = jnp.dot(hcol, as_ref[hd, :],
                         preferred_element_type=jnp.float32)
            vd = jnp.dot(hcol, ad_ref[hd, :],
                         preferred_element_type=jnp.float32)
            at_ref[:, hd:hd + 1] = va[:, None]
            dt_ref[:, hd:hd + 1] = vd[:, None]
            ms_ref[hd, :] = jnp.maximum(ms_ref[hd, :], jnp.max(va))
            md_ref[hd, :] = jnp.maximum(md_ref[hd, :], jnp.max(vd))

    return pl.pallas_call(
        body,
        grid=(N // BLK,),
        in_specs=[
            pl.BlockSpec((BLK, F_IN), lambda i: (i, 0)),
            pl.BlockSpec((F_IN, H * HID), lambda i: (0, 0)),
            pl.BlockSpec((H, HID), lambda i: (0, 0)),
            pl.BlockSpec((H, HID), lambda i: (0, 0)),
        ],
        out_specs=[pl.BlockSpec((BLK, 64), lambda i: (i, 0))
                   for _ in range(8)] +
                  [pl.BlockSpec((BLK, H), lambda i: (i, 0)),
                   pl.BlockSpec((BLK, H), lambda i: (i, 0)),
                   pl.BlockSpec((H, 128), lambda i: (0, 0)),
                   pl.BlockSpec((H, 128), lambda i: (0, 0))],
        out_shape=[jax.ShapeDtypeStruct((N, 64), jnp.float32)
                   for _ in range(8)] +
                  [jax.ShapeDtypeStruct((N, H), jnp.float32),
                   jax.ShapeDtypeStruct((N, H), jnp.float32),
                   jax.ShapeDtypeStruct((H, 128), jnp.float32),
                   jax.ShapeDtypeStruct((H, 128), jnp.float32)],
    )(x, W1, att_src1, att_dst1)


def _t5(nums, hs, s1parts, asrc, adst, bb, b1, W2, att_src2, att_dst2):
    BLK = 1000

    def body(*refs):
        nrefs = refs[:8]
        hrefs = refs[8:16]
        (s1_ref, at_ref, dt_ref, bb_ref, b1_ref, w2_ref, as2_ref, ad2_ref,
         hg_ref, a2_ref, d2_ref, m2s_ref, m2d_ref) = refs[16:]
        i = pl.program_id(0)
        cols = []
        for hd in range(H):
            exs = jnp.exp(_leaky(at_ref[:, hd:hd + 1] + dt_ref[:, hd:hd + 1])
                          - bb_ref[hd, 0])
            den = (s1_ref[hd % 2, :, hd // 2:hd // 2 + 1] + exs + 1e-16)
            cols.append((nrefs[hd][...] + exs * hrefs[hd][...]) / den)
        out1 = jnp.concatenate(cols, axis=1) + b1_ref[0, :]
        hh = jnp.where(out1 > 0, out1, jnp.exp(jnp.minimum(out1, 0.0)) - 1.0)
        hg = jnp.dot(hh, w2_ref[...], preferred_element_type=jnp.float32)
        a2 = jnp.dot(hg, as2_ref[0, :], preferred_element_type=jnp.float32)
        d2 = jnp.dot(hg, ad2_ref[0, :], preferred_element_type=jnp.float32)
        hg_ref[...] = hg
        a2_ref[...] = a2[:, None]
        d2_ref[...] = d2[:, None]

        @pl.when(i == 0)
        def _():
            m2s_ref[...] = jnp.full((8, 128), -jnp.inf, jnp.float32)
            md = jnp.full((8, 128), -jnp.inf, jnp.float32)
            m2d_ref[...] = md

        m2s_ref[...] = jnp.maximum(m2s_ref[...], jnp.max(a2))
        m2d_ref[...] = jnp.maximum(m2d_ref[...], jnp.max(d2))

    return pl.pallas_call(
        body,
        grid=(N // BLK,),
        in_specs=[pl.BlockSpec((BLK, 64), lambda i: (i, 0))
                  for _ in range(8)] +
                 [pl.BlockSpec((BLK, 64), lambda i: (i, 0))
                  for _ in range(8)] +
                 [pl.BlockSpec((2, BLK, 4), lambda i: (0, i, 0)),
                  pl.BlockSpec((BLK, H), lambda i: (i, 0)),
                  pl.BlockSpec((BLK, H), lambda i: (i, 0)),
                  pl.BlockSpec((H, 16), lambda i: (0, 0)),
                  pl.BlockSpec((1, 512), lambda i: (0, 0)),
                  pl.BlockSpec((512, CLS), lambda i: (0, 0)),
                  pl.BlockSpec((1, CLS), lambda i: (0, 0)),
                  pl.BlockSpec((1, CLS), lambda i: (0, 0))],
        out_specs=[pl.BlockSpec((BLK, CLS), lambda i: (i, 0)),
                   pl.BlockSpec((BLK, 1), lambda i: (i, 0)),
                   pl.BlockSpec((BLK, 1), lambda i: (i, 0)),
                   pl.BlockSpec((8, 128), lambda i: (0, 0)),
                   pl.BlockSpec((8, 128), lambda i: (0, 0))],
        out_shape=[jax.ShapeDtypeStruct((N, CLS), jnp.float32),
                   jax.ShapeDtypeStruct((N, 1), jnp.float32),
                   jax.ShapeDtypeStruct((N, 1), jnp.float32),
                   jax.ShapeDtypeStruct((8, 128), jnp.float32),
                   jax.ShapeDtypeStruct((8, 128), jnp.float32)],
    )(*nums, *hs, s1parts, asrc, adst, bb, b1.reshape(1, 512), W2,
      att_src2, att_dst2)


def _t6(num2, s2, h2g, asrc2, adst2, b2b, b2):
    BLK = 2000

    def body(n_ref, s_ref, hg_ref, a2_ref, d2_ref, bb_ref, b2_ref, out_ref):
        exs = jnp.exp(_leaky(a2_ref[...] + d2_ref[...]) - bb_ref[0, 0])
        den = s_ref[...] + exs + 1e-16
        out_ref[...] = (n_ref[...] + exs * hg_ref[...]) / den + b2_ref[0, :]

    return pl.pallas_call(
        body,
        grid=(N // BLK,),
        in_specs=[
            pl.BlockSpec((BLK, CLS), lambda i: (i, 0)),
            pl.BlockSpec((BLK, 1), lambda i: (i, 0)),
            pl.BlockSpec((BLK, CLS), lambda i: (i, 0)),
            pl.BlockSpec((BLK, 1), lambda i: (i, 0)),
            pl.BlockSpec((BLK, 1), lambda i: (i, 0)),
            pl.BlockSpec((1, 16), lambda i: (0, 0)),
            pl.BlockSpec((1, CLS), lambda i: (0, 0)),
        ],
        out_specs=pl.BlockSpec((BLK, CLS), lambda i: (i, 0)),
        out_shape=jax.ShapeDtypeStruct((N, CLS), jnp.float32),
    )(num2, s2, h2g, asrc2, adst2, b2b, b2.reshape(1, CLS))


def kernel(x, edge_index, dual_x, dual_edge_index, epoch,
           W1, att_src1, att_dst1, b1, W2, att_src2, att_dst2, b2,
           Wg1, bg1, Wg2, bg2):
    f32 = jnp.float32
    i32 = jnp.int32

    dsrc = jnp.concatenate(
        [dual_edge_index[0], jnp.zeros((EDP - ED,), i32)])
    ddst = jnp.concatenate(
        [dual_edge_index[1], jnp.full((EDP - ED,), ND, i32)])
    psrc = jnp.concatenate([edge_index[0], jnp.zeros((EP - E,), i32)])
    pdst = jnp.concatenate([edge_index[1], jnp.full((EP - E,), N, i32)])

    ones128 = jnp.ones((128,), f32)
    z1d = jnp.zeros((2048,), f32)
    z2d16 = jnp.zeros((128, 16), f32)
    z2d64 = jnp.zeros((128, 64), f32)
    z632 = jnp.zeros((632,), f32)

    histp = _sc_hist(ddst, ones128, z1d)
    deg = histp[:ND] + 1.0
    dinv2d = lax.rsqrt(deg)[:, None]
    g1 = _t1(dual_x, Wg1, dinv2d)
    s1 = [_sc_segsum(dsrc, ddst, g1[j], z2d16)[:ND] for j in range(4)]
    g2f = _t2(dual_x, Wg1, Wg2, bg1, dinv2d, s1)
    s2 = _sc_segsum(dsrc, ddst, g2f, z2d16)[:ND]
    Q = _t3(dinv2d, s2, g2f, bg2)

    t4 = _t4(x, W1, att_src1, att_dst1)
    hs = list(t4[0:8])
    asrc, adst, msrc, mdst = t4[8], t4[9], t4[10], t4[11]
    B = _leaky(jnp.max(msrc, axis=1) + jnp.max(mdst, axis=1))
    bb = jnp.broadcast_to(B[:, None], (H, 16)).astype(f32)
    asrcTp = jnp.pad(asrc.T, ((0, 0), (0, NP - N))).reshape(H * NP)
    adstTp = jnp.pad(adst.T, ((0, 0), (0, NP - N))).reshape(H * NP)
    ex1, s1flat = _sc_gat1_attn(psrc, pdst, asrcTp, adstTp, bb, z1d)
    s1parts = s1flat.reshape(2, NP, 4)[:, :N, :]
    nums = _sc_gat1_num(psrc, pdst, ex1, hs, z2d64)
    nums = [a[:N] for a in nums]
    h2g, asrc2, adst2, m2s, m2d = _t5(
        nums, hs, s1parts, asrc, adst, bb, b1, W2, att_src2, att_dst2)
    B2 = _leaky(jnp.max(m2s) + jnp.max(m2d))
    b2b = jnp.broadcast_to(B2, (1, 16)).astype(f32)
    a2p = jnp.pad(asrc2.reshape(N), (0, NP - N))
    d2p = jnp.pad(adst2.reshape(N), (0, NP - N))
    num2, s2g = _sc_gat2(psrc, pdst, a2p, d2p, b2b, h2g, z2d16, z632)
    out = _t6(num2[:N], s2g[:N].reshape(N, 1), h2g, asrc2, adst2, b2b, b2)
    return (out, Q)
